# baseline (speedup 1.0000x reference)
import functools

import jax
import jax.numpy as jnp
from jax import lax
from jax.experimental import pallas as pl
from jax.experimental.pallas import tpu as pltpu
from jax.experimental.pallas import tpu_sc as plsc

FEAT_STRIDE = 16.0
PRE_NMS_TOPN = 2000
POST_NMS_TOPN = 300
NMS_THRESH = 0.7
MIN_SIZE = 16.0

_N = 2048
_BLK = 128
_OUT_ROWS = 384
_NEG = -1e9


def _transform_body(fg_ref, dl_ref, anch_ref, im_ref, x1_ref, y1_ref, x2_ref, y2_ref, sc_ref, key_ref):
    hw = jax.lax.broadcasted_iota(jnp.int32, (9, 2500), 1).astype(jnp.float32)
    row = jnp.floor((hw + 0.5) * (1.0 / 50.0))
    sy = row * FEAT_STRIDE
    sx = (hw - 50.0 * row) * FEAT_STRIDE

    ax1 = anch_ref[:, 0:1] + sx
    ay1 = anch_ref[:, 1:2] + sy
    ax2 = anch_ref[:, 2:3] + sx
    ay2 = anch_ref[:, 3:4] + sy

    widths = ax2 - ax1 + 1.0
    heights = ay2 - ay1 + 1.0
    ctr_x = ax1 + 0.5 * widths
    ctr_y = ay1 + 0.5 * heights

    dx = dl_ref[:, 0, :]
    dy = dl_ref[:, 1, :]
    dw = dl_ref[:, 2, :]
    dh = dl_ref[:, 3, :]

    pred_ctr_x = dx * widths + ctr_x
    pred_ctr_y = dy * heights + ctr_y
    pred_w = jnp.exp(dw) * widths
    pred_h = jnp.exp(dh) * heights

    im_h = im_ref[0:1, 0:1]
    im_w = im_ref[0:1, 1:2]
    im_scale = im_ref[0:1, 2:3]

    x1 = jnp.clip(pred_ctr_x - 0.5 * pred_w, 0.0, im_w - 1.0)
    y1 = jnp.clip(pred_ctr_y - 0.5 * pred_h, 0.0, im_h - 1.0)
    x2 = jnp.clip(pred_ctr_x + 0.5 * pred_w, 0.0, im_w - 1.0)
    y2 = jnp.clip(pred_ctr_y + 0.5 * pred_h, 0.0, im_h - 1.0)

    ws = x2 - x1 + 1.0
    hs = y2 - y1 + 1.0
    min_size = MIN_SIZE * im_scale
    valid = (ws >= min_size) & (hs >= min_size)

    x1_ref[...] = x1
    y1_ref[...] = y1
    x2_ref[...] = x2
    y2_ref[...] = y2
    scm = jnp.where(valid, fg_ref[...], _NEG)
    sc_ref[...] = scm
    b = jax.lax.bitcast_convert_type(scm, jnp.int32)
    u = jnp.where(b >= 0, b ^ jnp.int32(-2147483648), ~b)
    key_ref[...] = ~u


def _transform(fg, dl, anchors, im_info):
    out = jax.ShapeDtypeStruct((9, 2500), jnp.float32)
    outi = jax.ShapeDtypeStruct((9, 2500), jnp.int32)
    return pl.pallas_call(
        _transform_body,
        out_shape=(out, out, out, out, out, outi),
    )(fg, dl, anchors, im_info)


_NALL = 22528
_NTILES = 16
_CHUNK = _NALL // _NTILES
_CVECS = _CHUNK // 16
_NSORT = 2048
_LPB = _NSORT // 16


def _sc_topk_body(k2_hbm, table_hbm, out_hbm,
                  lock2, hist, dig, gh, stmp, allk2,
                  ka, na, kb, nb, scan4096, idx2d, rows, shist, sem):
    cid = lax.axis_index("c")
    sid = lax.axis_index("s")
    lane = lax.broadcasted_iota(jnp.int32, (16,), 0)
    ones = jnp.ones((16,), jnp.int32)
    zeros16 = jnp.zeros((16,), jnp.int32)

    @pl.when(cid == 0)
    def _core0():
        base = sid * _CHUNK
        pltpu.sync_copy(k2_hbm.at[pl.ds(base, _CHUNK)], lock2)

        r = jnp.int32(PRE_NMS_TOPN)
        p = jnp.int32(0)
        for rnd in range(4):
            shift = 24 - 8 * rnd

            def zero_h(v, _):
                hist[pl.ds(v * 16, 16)] = zeros16
                return 0
            lax.fori_loop(0, 256, zero_h, 0)

            p_u = p.astype(jnp.uint32)

            def hist_body(v, _):
                kv = lock2[pl.ds(v * 16, 16)]
                ku = plsc.bitcast(kv, jnp.uint32)
                d = (lax.shift_right_logical(ku, jnp.uint32(shift))
                     & jnp.uint32(255)).astype(jnp.int32)
                if rnd == 0:
                    plsc.addupdate_scatter(hist, [lane * 256 + d], ones)
                else:
                    pm = lax.shift_right_logical(ku, jnp.uint32(shift + 8)) == p_u
                    plsc.addupdate_scatter(hist, [lane * 256 + d], ones, mask=pm)
                return 0
            lax.fori_loop(0, _CVECS, hist_body, 0)

            for c in range(16):
                acc = hist[pl.ds(c * 16, 16)]
                for l in range(1, 16):
                    acc = acc + hist[pl.ds(l * 256 + c * 16, 16)]
                dig[pl.ds(c * 16, 16)] = acc
            pltpu.sync_copy(dig, shist.at[rnd, sid])
            plsc.subcore_barrier()
            pltpu.sync_copy(shist.at[rnd], gh)

            running = jnp.int32(0)
            nless = jnp.int32(0)
            for c in range(16):
                s = gh[0, pl.ds(c * 16, 16)]
                for t in range(1, _NTILES):
                    s = s + gh[t, pl.ds(c * 16, 16)]
                cum = plsc.cumsum(s) + running
                stmp[pl.ds(c * 16, 16)] = cum
                running = running + jnp.sum(s)
                nless = nless + jnp.sum((cum < r).astype(jnp.int32))
            bstar = nless
            gidx = jnp.maximum(bstar - 1, 0) + zeros16
            sbm1 = jnp.max(plsc.load_gather(stmp, [gidx]))
            sbm1 = jnp.where(bstar > 0, sbm1, 0)
            r = r - sbm1
            p = (p << 8) | bstar
        tx = p ^ jnp.int32(-2147483648)
        tv = p
        r_eq = r

        @pl.when(sid == 0)
        def _tile0():
            pltpu.sync_copy(k2_hbm, allk2)

            def compact(v, carry):
                cnt, eqc = carry
                kv = allk2[pl.ds(v * 16, 16)]
                lt = (kv ^ jnp.int32(-2147483648)) < tx
                eq = kv == tv
                eqi = eq.astype(jnp.int32)
                eqrank = eqc + plsc.cumsum(eqi) - 1
                take = lt | (eq & (eqrank < r_eq))
                ti = take.astype(jnp.int32)
                posv = cnt + plsc.cumsum(ti) - 1
                plsc.store_scatter(ka, [posv], kv, mask=take)
                plsc.store_scatter(na, [posv], v * 16 + lane, mask=take)
                return cnt + jnp.sum(ti), eqc + jnp.sum(eqi)
            lax.fori_loop(0, _NALL // 16, compact, (jnp.int32(0), jnp.int32(0)))

            for v in range(PRE_NMS_TOPN // 16, _LPB):
                ka[pl.ds(v * 16, 16)] = jnp.full((16,), -1, jnp.int32)
                na[pl.ds(v * 16, 16)] = jnp.full((16,), _NALL - 1, jnp.int32)

            bufs = [(ka, na), (kb, nb)]
            for pno in range(4):
                shift = 8 * pno
                src_k, src_n = bufs[pno % 2]
                dst_k, dst_n = bufs[(pno + 1) % 2]

                def zero_h2(v, _):
                    hist[pl.ds(v * 16, 16)] = zeros16
                    return 0
                lax.fori_loop(0, 256, zero_h2, 0)

                def hist2(v, _):
                    kv = plsc.load_gather(src_k, [lane * _LPB + v])
                    ku = plsc.bitcast(kv, jnp.uint32)
                    d = (lax.shift_right_logical(ku, jnp.uint32(shift))
                         & jnp.uint32(255)).astype(jnp.int32)
                    plsc.addupdate_scatter(hist, [d * 16 + lane], ones)
                    return 0
                lax.fori_loop(0, _LPB, hist2, 0)

                def scan_b(c, running):
                    h = hist[pl.ds(c * 16, 16)]
                    scan4096[pl.ds(c * 16, 16)] = plsc.cumsum(h) - h + running
                    return running + jnp.sum(h)
                lax.fori_loop(0, 256, scan_b, jnp.int32(0))

                def reorder(v, _):
                    eidx = lane * _LPB + v
                    kv = plsc.load_gather(src_k, [eidx])
                    nv = plsc.load_gather(src_n, [eidx])
                    ku = plsc.bitcast(kv, jnp.uint32)
                    d = (lax.shift_right_logical(ku, jnp.uint32(shift))
                         & jnp.uint32(255)).astype(jnp.int32)
                    hidx = d * 16 + lane
                    a = plsc.load_gather(scan4096, [hidx])
                    plsc.store_scatter(dst_k, [a], kv)
                    plsc.store_scatter(dst_n, [a], nv)
                    plsc.addupdate_scatter(scan4096, [hidx], ones)
                    return 0
                lax.fori_loop(0, _LPB, reorder, 0)

            for i in range(16):
                for u in range(8):
                    idx2d[i, pl.ds(u * 16, 16)] = na[pl.ds(i * 128 + u * 16, 16)]
            for i in range(16):
                pltpu.async_copy(table_hbm.at[idx2d.at[i]],
                                 rows.at[pl.ds(i * 128, 128)], sem).wait()
            zf = jnp.zeros((16,), jnp.float32)
            for j in range(PRE_NMS_TOPN, _NSORT):
                rows[j, :] = zf
            pltpu.sync_copy(rows, out_hbm)


@functools.partial(
    pl.kernel,
    out_type=jax.ShapeDtypeStruct((_NSORT, 16), jnp.float32),
    mesh=plsc.VectorSubcoreMesh(core_axis_name="c", subcore_axis_name="s"),
    compiler_params=pltpu.CompilerParams(needs_layout_passes=False,
                                         use_tc_tiling_on_sc=False),
    scratch_types=[
        pltpu.VMEM((_CHUNK,), jnp.int32),
        pltpu.VMEM((4096,), jnp.int32),
        pltpu.VMEM((256,), jnp.int32),
        pltpu.VMEM((_NTILES, 256), jnp.int32),
        pltpu.VMEM((256,), jnp.int32),
        pltpu.VMEM((_NALL,), jnp.int32),
        pltpu.VMEM((_NSORT,), jnp.int32),
        pltpu.VMEM((_NSORT,), jnp.int32),
        pltpu.VMEM((_NSORT,), jnp.int32),
        pltpu.VMEM((_NSORT,), jnp.int32),
        pltpu.VMEM((4096,), jnp.int32),
        pltpu.VMEM((16, 128), jnp.int32),
        pltpu.VMEM((_NSORT, 16), jnp.float32),
        pltpu.VMEM_SHARED((4, _NTILES, 256), jnp.int32),
        pltpu.SemaphoreType.DMA,
    ],
)
def _sc_topk(k2_hbm, table_hbm, out_hbm, *scratch):
    _sc_topk_body(k2_hbm, table_hbm, out_hbm, *scratch)


def _nms_body(rows_ref, cols_ref, out_ref, s_mat, lt_mat):
    x1r = rows_ref[0:1, :]
    y1r = rows_ref[1:2, :]
    x2r = rows_ref[2:3, :]
    y2r = rows_ref[3:4, :]
    area_r = (x2r - x1r + 1.0) * (y2r - y1r + 1.0)

    for b in range(_N // _BLK):
        sl = pl.ds(b * _BLK, _BLK)
        x1c = cols_ref[sl, 0:1]
        y1c = cols_ref[sl, 1:2]
        x2c = cols_ref[sl, 2:3]
        y2c = cols_ref[sl, 3:4]
        area_c = (x2c - x1c + 1.0) * (y2c - y1c + 1.0)
        xx1 = jnp.maximum(x1c, x1r)
        yy1 = jnp.maximum(y1c, y1r)
        xx2 = jnp.minimum(x2c, x2r)
        yy2 = jnp.minimum(y2c, y2r)
        inter = jnp.maximum(xx2 - xx1 + 1.0, 0.0) * jnp.maximum(yy2 - yy1 + 1.0, 0.0)
        iou = inter / (area_c + area_r - inter)
        jg = b * _BLK + jax.lax.broadcasted_iota(jnp.int32, (_BLK, _N), 0)
        ig = jax.lax.broadcasted_iota(jnp.int32, (_BLK, _N), 1)
        sup = (iou > NMS_THRESH) & (jg < ig) & (ig < PRE_NMS_TOPN) & (jg < PRE_NMS_TOPN)
        s_mat[sl, :] = sup.astype(jnp.bfloat16)
        lt_mat[sl, :] = (jg <= ig).astype(jnp.bfloat16)

    icol = jax.lax.broadcasted_iota(jnp.int32, (8, _N), 1)
    inb = (icol < PRE_NMS_TOPN).astype(jnp.float32)
    keep0 = inb

    def cond(carry):
        _, changed, it = carry
        return changed & (it < _N)

    def body(carry):
        keep, _, it = carry
        sup = jnp.dot(keep.astype(jnp.bfloat16), s_mat[...],
                      preferred_element_type=jnp.float32)
        nk = jnp.where((sup < 0.5) & (icol < PRE_NMS_TOPN), 1.0, 0.0)
        changed = jnp.sum(jnp.abs(nk - keep)) > 0.0
        return nk, changed, it + 1

    keep, _, _ = jax.lax.while_loop(cond, body, (keep0, True, 0))

    kcount = jnp.sum(keep[0:1, :])
    fill = (1.0 - keep) * inb
    cumk = jnp.dot(keep.astype(jnp.bfloat16), lt_mat[...],
                   preferred_element_type=jnp.float32)
    cumf = jnp.dot(fill.astype(jnp.bfloat16), lt_mat[...],
                   preferred_element_type=jnp.float32)
    pos = jnp.where(keep > 0.5, cumk - 1.0, kcount + cumf - 1.0)
    pos = jnp.where(inb > 0.5, pos, 1e9)
    pos1 = pos[0:1, :]

    riota = jax.lax.broadcasted_iota(jnp.int32, (_OUT_ROWS, _N), 0).astype(jnp.float32)
    posb = jnp.broadcast_to(pos1, (_OUT_ROWS, _N))
    sel = riota == posb

    def pick(vals_row):
        v = jnp.broadcast_to(vals_row, (_OUT_ROWS, _N))
        return jnp.max(jnp.where(sel, v, -3.0e38), axis=1, keepdims=True)

    rvec = jax.lax.broadcasted_iota(jnp.int32, (_OUT_ROWS, 1), 0).astype(jnp.float32)
    out_ref[:, 0:1] = jnp.zeros((_OUT_ROWS, 1), jnp.float32)
    out_ref[:, 1:2] = pick(x1r)
    out_ref[:, 2:3] = pick(y1r)
    out_ref[:, 3:4] = pick(x2r)
    out_ref[:, 4:5] = pick(y2r)
    scpick = pick(rows_ref[4:5, :])
    out_ref[:, 5:6] = jnp.where(rvec < kcount, scpick, _NEG)
    out_ref[:, 6:8] = jnp.zeros((_OUT_ROWS, 2), jnp.float32)


def _nms(rows, cols):
    return pl.pallas_call(
        _nms_body,
        out_shape=jax.ShapeDtypeStruct((_OUT_ROWS, 8), jnp.float32),
        scratch_shapes=[
            pltpu.VMEM((_N, _N), jnp.bfloat16),
            pltpu.VMEM((_N, _N), jnp.bfloat16),
        ],
    )(rows, cols)


def kernel(scores, bbox_deltas, im_info, anchors):
    A = anchors.shape[0]
    H, W = scores.shape[2], scores.shape[3]
    fg = scores[0, A:].reshape(A, H * W)
    dl = bbox_deltas[0].reshape(A, 4, H * W)

    x1, y1, x2, y2, scm, key = _transform(fg, dl, anchors, im_info)

    def flat(v):
        return v.T.reshape(-1)

    npad = _NALL - A * H * W
    k2 = jnp.concatenate([flat(key), jnp.full((npad,), -1, jnp.int32)])
    tab = jnp.stack([flat(x1), flat(y1), flat(x2), flat(y2), flat(scm)], axis=1)
    tab = jnp.pad(tab, ((0, npad), (0, 11)))

    sorted_tab = _sc_topk(k2, tab)
    cols = sorted_tab[:, :8]
    rows = cols.T

    out = _nms(rows, cols)
    return out[:POST_NMS_TOPN, :6]


# speedup vs baseline: 1.0945x; 1.0945x over previous
import functools

import jax
import jax.numpy as jnp
from jax import lax
from jax.experimental import pallas as pl
from jax.experimental.pallas import tpu as pltpu
from jax.experimental.pallas import tpu_sc as plsc

FEAT_STRIDE = 16.0
PRE_NMS_TOPN = 2000
POST_NMS_TOPN = 300
NMS_THRESH = 0.7
MIN_SIZE = 16.0

_N = 2048
_BLK = 128
_OUT_ROWS = 384
_NEG = -1e9


def _transform_body(fg_ref, dl_ref, anch_ref, im_ref, x1_ref, y1_ref, x2_ref, y2_ref, sc_ref, key_ref):
    hw = jax.lax.broadcasted_iota(jnp.int32, (9, 2500), 1).astype(jnp.float32)
    row = jnp.floor((hw + 0.5) * (1.0 / 50.0))
    sy = row * FEAT_STRIDE
    sx = (hw - 50.0 * row) * FEAT_STRIDE

    ax1 = anch_ref[:, 0:1] + sx
    ay1 = anch_ref[:, 1:2] + sy
    ax2 = anch_ref[:, 2:3] + sx
    ay2 = anch_ref[:, 3:4] + sy

    widths = ax2 - ax1 + 1.0
    heights = ay2 - ay1 + 1.0
    ctr_x = ax1 + 0.5 * widths
    ctr_y = ay1 + 0.5 * heights

    dx = dl_ref[:, 0, :]
    dy = dl_ref[:, 1, :]
    dw = dl_ref[:, 2, :]
    dh = dl_ref[:, 3, :]

    pred_ctr_x = dx * widths + ctr_x
    pred_ctr_y = dy * heights + ctr_y
    pred_w = jnp.exp(dw) * widths
    pred_h = jnp.exp(dh) * heights

    im_h = im_ref[0:1, 0:1]
    im_w = im_ref[0:1, 1:2]
    im_scale = im_ref[0:1, 2:3]

    x1 = jnp.clip(pred_ctr_x - 0.5 * pred_w, 0.0, im_w - 1.0)
    y1 = jnp.clip(pred_ctr_y - 0.5 * pred_h, 0.0, im_h - 1.0)
    x2 = jnp.clip(pred_ctr_x + 0.5 * pred_w, 0.0, im_w - 1.0)
    y2 = jnp.clip(pred_ctr_y + 0.5 * pred_h, 0.0, im_h - 1.0)

    ws = x2 - x1 + 1.0
    hs = y2 - y1 + 1.0
    min_size = MIN_SIZE * im_scale
    valid = (ws >= min_size) & (hs >= min_size)

    x1_ref[...] = x1
    y1_ref[...] = y1
    x2_ref[...] = x2
    y2_ref[...] = y2
    scm = jnp.where(valid, fg_ref[...], _NEG)
    sc_ref[...] = scm
    b = jax.lax.bitcast_convert_type(scm, jnp.int32)
    u = jnp.where(b >= 0, b ^ jnp.int32(-2147483648), ~b)
    key_ref[...] = ~u


def _transform(fg, dl, anchors, im_info):
    out = jax.ShapeDtypeStruct((9, 2500), jnp.float32)
    outi = jax.ShapeDtypeStruct((9, 2500), jnp.int32)
    return pl.pallas_call(
        _transform_body,
        out_shape=(out, out, out, out, out, outi),
    )(fg, dl, anchors, im_info)


_NALL = 22528
_NTILES = 16
_CHUNK = _NALL // _NTILES
_CVECS = _CHUNK // 16
_NSORT = 2048
_LPB = _NSORT // 16


def _sc_topk_body(k2_hbm, table_hbm, out_hbm,
                  lock2, hist, dig, gh, stmp, allk2,
                  ka, na, kb, nb, scan4096, idx2d, rows, shist, sem):
    cid = lax.axis_index("c")
    sid = lax.axis_index("s")
    lane = lax.broadcasted_iota(jnp.int32, (16,), 0)
    ones = jnp.ones((16,), jnp.int32)
    zeros16 = jnp.zeros((16,), jnp.int32)

    @pl.when(cid == 0)
    def _core0():
        base = sid * _CHUNK
        pltpu.sync_copy(k2_hbm.at[pl.ds(base, _CHUNK)], lock2)

        r = jnp.int32(PRE_NMS_TOPN)
        p = jnp.int32(0)
        for rnd in range(4):
            shift = 24 - 8 * rnd

            def zero_h(v, _):
                hist[pl.ds(v * 16, 16)] = zeros16
                return 0
            lax.fori_loop(0, 256, zero_h, 0)

            p_u = p.astype(jnp.uint32)

            def hist_body(v, _):
                kv = lock2[pl.ds(v * 16, 16)]
                ku = plsc.bitcast(kv, jnp.uint32)
                d = (lax.shift_right_logical(ku, jnp.uint32(shift))
                     & jnp.uint32(255)).astype(jnp.int32)
                if rnd == 0:
                    plsc.addupdate_scatter(hist, [lane * 256 + d], ones)
                else:
                    pm = lax.shift_right_logical(ku, jnp.uint32(shift + 8)) == p_u
                    plsc.addupdate_scatter(hist, [lane * 256 + d], ones, mask=pm)
                return 0
            lax.fori_loop(0, _CVECS, hist_body, 0)

            for c in range(16):
                acc = hist[pl.ds(c * 16, 16)]
                for l in range(1, 16):
                    acc = acc + hist[pl.ds(l * 256 + c * 16, 16)]
                dig[pl.ds(c * 16, 16)] = acc
            pltpu.sync_copy(dig, shist.at[rnd, sid])
            plsc.subcore_barrier()
            pltpu.sync_copy(shist.at[rnd], gh)

            running = jnp.int32(0)
            nless = jnp.int32(0)
            for c in range(16):
                s = gh[0, pl.ds(c * 16, 16)]
                for t in range(1, _NTILES):
                    s = s + gh[t, pl.ds(c * 16, 16)]
                cum = plsc.cumsum(s) + running
                stmp[pl.ds(c * 16, 16)] = cum
                running = running + jnp.sum(s)
                nless = nless + jnp.sum((cum < r).astype(jnp.int32))
            bstar = nless
            gidx = jnp.maximum(bstar - 1, 0) + zeros16
            sbm1 = jnp.max(plsc.load_gather(stmp, [gidx]))
            sbm1 = jnp.where(bstar > 0, sbm1, 0)
            r = r - sbm1
            p = (p << 8) | bstar
        tx = p ^ jnp.int32(-2147483648)
        tv = p
        r_eq = r

        @pl.when(sid == 0)
        def _tile0():
            pltpu.sync_copy(k2_hbm, allk2)

            def compact(v, carry):
                cnt, eqc = carry
                kv = allk2[pl.ds(v * 16, 16)]
                lt = (kv ^ jnp.int32(-2147483648)) < tx
                eq = kv == tv
                eqi = eq.astype(jnp.int32)
                eqrank = eqc + plsc.cumsum(eqi) - 1
                take = lt | (eq & (eqrank < r_eq))
                ti = take.astype(jnp.int32)
                posv = cnt + plsc.cumsum(ti) - 1
                plsc.store_scatter(ka, [posv], kv, mask=take)
                plsc.store_scatter(na, [posv], v * 16 + lane, mask=take)
                return cnt + jnp.sum(ti), eqc + jnp.sum(eqi)
            lax.fori_loop(0, _NALL // 16, compact, (jnp.int32(0), jnp.int32(0)))

            for v in range(PRE_NMS_TOPN // 16, _LPB):
                ka[pl.ds(v * 16, 16)] = jnp.full((16,), -1, jnp.int32)
                na[pl.ds(v * 16, 16)] = jnp.full((16,), _NALL - 1, jnp.int32)

            bufs = [(ka, na), (kb, nb)]
            for pno in range(4):
                shift = 8 * pno
                src_k, src_n = bufs[pno % 2]
                dst_k, dst_n = bufs[(pno + 1) % 2]

                def zero_h2(v, _):
                    hist[pl.ds(v * 16, 16)] = zeros16
                    return 0
                lax.fori_loop(0, 256, zero_h2, 0)

                def hist2(v, _):
                    kv = plsc.load_gather(src_k, [lane * _LPB + v])
                    ku = plsc.bitcast(kv, jnp.uint32)
                    d = (lax.shift_right_logical(ku, jnp.uint32(shift))
                         & jnp.uint32(255)).astype(jnp.int32)
                    plsc.addupdate_scatter(hist, [d * 16 + lane], ones)
                    return 0
                lax.fori_loop(0, _LPB, hist2, 0)

                def scan_b(c, running):
                    h = hist[pl.ds(c * 16, 16)]
                    scan4096[pl.ds(c * 16, 16)] = plsc.cumsum(h) - h + running
                    return running + jnp.sum(h)
                lax.fori_loop(0, 256, scan_b, jnp.int32(0))

                def reorder(v, _):
                    eidx = lane * _LPB + v
                    kv = plsc.load_gather(src_k, [eidx])
                    nv = plsc.load_gather(src_n, [eidx])
                    ku = plsc.bitcast(kv, jnp.uint32)
                    d = (lax.shift_right_logical(ku, jnp.uint32(shift))
                         & jnp.uint32(255)).astype(jnp.int32)
                    hidx = d * 16 + lane
                    a = plsc.load_gather(scan4096, [hidx])
                    plsc.store_scatter(dst_k, [a], kv)
                    plsc.store_scatter(dst_n, [a], nv)
                    plsc.addupdate_scatter(scan4096, [hidx], ones)
                    return 0
                lax.fori_loop(0, _LPB, reorder, 0)

            for i in range(16):
                for u in range(8):
                    idx2d[i, pl.ds(u * 16, 16)] = na[pl.ds(i * 128 + u * 16, 16)]
            for i in range(16):
                pltpu.async_copy(table_hbm.at[idx2d.at[i]],
                                 rows.at[pl.ds(i * 128, 128)], sem).wait()
            zf = jnp.zeros((16,), jnp.float32)
            for j in range(PRE_NMS_TOPN, _NSORT):
                rows[j, :] = zf
            pltpu.sync_copy(rows, out_hbm)


@functools.cache
def _make_sc_topk():
    @functools.partial(
        pl.kernel,
        out_type=jax.ShapeDtypeStruct((_NSORT, 16), jnp.float32),
        mesh=plsc.VectorSubcoreMesh(core_axis_name="c", subcore_axis_name="s"),
        compiler_params=pltpu.CompilerParams(needs_layout_passes=False,
                                             use_tc_tiling_on_sc=False),
        scratch_types=[
            pltpu.VMEM((_CHUNK,), jnp.int32),
            pltpu.VMEM((4096,), jnp.int32),
            pltpu.VMEM((256,), jnp.int32),
            pltpu.VMEM((_NTILES, 256), jnp.int32),
            pltpu.VMEM((256,), jnp.int32),
            pltpu.VMEM((_NALL,), jnp.int32),
            pltpu.VMEM((_NSORT,), jnp.int32),
            pltpu.VMEM((_NSORT,), jnp.int32),
            pltpu.VMEM((_NSORT,), jnp.int32),
            pltpu.VMEM((_NSORT,), jnp.int32),
            pltpu.VMEM((4096,), jnp.int32),
            pltpu.VMEM((16, 128), jnp.int32),
            pltpu.VMEM((_NSORT, 16), jnp.float32),
            pltpu.VMEM_SHARED((4, _NTILES, 256), jnp.int32),
            pltpu.SemaphoreType.DMA,
        ],
    )
    def _sc_topk(k2_hbm, table_hbm, out_hbm, *scratch):
        _sc_topk_body(k2_hbm, table_hbm, out_hbm, *scratch)

    return _sc_topk


_P1 = 512


def _nms_block(rows4, cols_ref, b, ncols):
    x1r, y1r, x2r, y2r, area_r = rows4
    sl = pl.ds(b * _BLK, _BLK)
    x1c = cols_ref[sl, 0:1]
    y1c = cols_ref[sl, 1:2]
    x2c = cols_ref[sl, 2:3]
    y2c = cols_ref[sl, 3:4]
    area_c = (x2c - x1c + 1.0) * (y2c - y1c + 1.0)
    xx1 = jnp.maximum(x1c, x1r[:, :ncols])
    yy1 = jnp.maximum(y1c, y1r[:, :ncols])
    xx2 = jnp.minimum(x2c, x2r[:, :ncols])
    yy2 = jnp.minimum(y2c, y2r[:, :ncols])
    inter = jnp.maximum(xx2 - xx1 + 1.0, 0.0) * jnp.maximum(yy2 - yy1 + 1.0, 0.0)
    iou = inter / (area_c + area_r[:, :ncols] - inter)
    jg = b * _BLK + jax.lax.broadcasted_iota(jnp.int32, (_BLK, ncols), 0)
    ig = jax.lax.broadcasted_iota(jnp.int32, (_BLK, ncols), 1)
    sup = (iou > NMS_THRESH) & (jg < ig) & (ig < PRE_NMS_TOPN) & (jg < PRE_NMS_TOPN)
    return sup.astype(jnp.bfloat16), (jg <= ig).astype(jnp.bfloat16)


def _nms_body(rows_ref, cols_ref, out_ref, s_mat, lt_mat, pos_ref, kc_ref):
    x1r = rows_ref[0:1, :]
    y1r = rows_ref[1:2, :]
    x2r = rows_ref[2:3, :]
    y2r = rows_ref[3:4, :]
    area_r = (x2r - x1r + 1.0) * (y2r - y1r + 1.0)
    rows4 = (x1r, y1r, x2r, y2r, area_r)

    for b in range(_P1 // _BLK):
        s_blk, lt_blk = _nms_block(rows4, cols_ref, b, _P1)
        s_mat[pl.ds(b * _BLK, _BLK), pl.ds(0, _P1)] = s_blk
        lt_mat[pl.ds(b * _BLK, _BLK), pl.ds(0, _P1)] = lt_blk

    icol1 = jax.lax.broadcasted_iota(jnp.int32, (8, _P1), 1)
    keep0a = jnp.ones((8, _P1), jnp.float32)

    def cond_a(carry):
        _, changed, it = carry
        return changed & (it < _P1)

    def body_a(carry):
        keep, _, it = carry
        sup = jnp.dot(keep.astype(jnp.bfloat16),
                      s_mat[pl.ds(0, _P1), pl.ds(0, _P1)],
                      preferred_element_type=jnp.float32)
        nk = jnp.where(sup < 0.5, 1.0, 0.0)
        changed = jnp.sum(jnp.abs(nk - keep)) > 0.0
        return nk, changed, it + 1

    keep_a, _, _ = jax.lax.while_loop(cond_a, body_a, (keep0a, True, 0))
    kept_a = jnp.sum(keep_a[0:1, :])

    @pl.when(kept_a >= float(POST_NMS_TOPN))
    def _fast():
        cumk = jnp.dot(keep_a.astype(jnp.bfloat16),
                       lt_mat[pl.ds(0, _P1), pl.ds(0, _P1)],
                       preferred_element_type=jnp.float32)
        pos_ref[:, 0:_P1] = jnp.where(keep_a > 0.5, cumk - 1.0, 1e9)
        pos_ref[:, _P1:] = jnp.full((8, _N - _P1), 1e9, jnp.float32)
        kc_ref[0] = kept_a

    @pl.when(kept_a < float(POST_NMS_TOPN))
    def _slow():
        for b in range(_N // _BLK):
            s_blk, lt_blk = _nms_block(rows4, cols_ref, b, _N)
            s_mat[pl.ds(b * _BLK, _BLK), :] = s_blk
            lt_mat[pl.ds(b * _BLK, _BLK), :] = lt_blk

        icol = jax.lax.broadcasted_iota(jnp.int32, (8, _N), 1)
        inb = (icol < PRE_NMS_TOPN).astype(jnp.float32)

        def cond(carry):
            _, changed, it = carry
            return changed & (it < _N)

        def body(carry):
            keep, _, it = carry
            sup = jnp.dot(keep.astype(jnp.bfloat16), s_mat[...],
                          preferred_element_type=jnp.float32)
            nk = jnp.where((sup < 0.5) & (icol < PRE_NMS_TOPN), 1.0, 0.0)
            changed = jnp.sum(jnp.abs(nk - keep)) > 0.0
            return nk, changed, it + 1

        keep, _, _ = jax.lax.while_loop(cond, body, (inb, True, 0))

        kcount = jnp.sum(keep[0:1, :])
        fill = (1.0 - keep) * inb
        cumk = jnp.dot(keep.astype(jnp.bfloat16), lt_mat[...],
                       preferred_element_type=jnp.float32)
        cumf = jnp.dot(fill.astype(jnp.bfloat16), lt_mat[...],
                       preferred_element_type=jnp.float32)
        pos = jnp.where(keep > 0.5, cumk - 1.0, kcount + cumf - 1.0)
        pos_ref[...] = jnp.where(inb > 0.5, pos, 1e9)
        kc_ref[0] = kcount

    kcount = kc_ref[0]
    pos1 = pos_ref[0:1, :]

    riota = jax.lax.broadcasted_iota(jnp.int32, (_OUT_ROWS, _N), 0).astype(jnp.float32)
    posb = jnp.broadcast_to(pos1, (_OUT_ROWS, _N))
    sel = riota == posb

    def pick(vals_row):
        v = jnp.broadcast_to(vals_row, (_OUT_ROWS, _N))
        return jnp.max(jnp.where(sel, v, -3.0e38), axis=1, keepdims=True)

    rvec = jax.lax.broadcasted_iota(jnp.int32, (_OUT_ROWS, 1), 0).astype(jnp.float32)
    out_ref[:, 0:1] = jnp.zeros((_OUT_ROWS, 1), jnp.float32)
    out_ref[:, 1:2] = pick(x1r)
    out_ref[:, 2:3] = pick(y1r)
    out_ref[:, 3:4] = pick(x2r)
    out_ref[:, 4:5] = pick(y2r)
    scpick = pick(rows_ref[4:5, :])
    out_ref[:, 5:6] = jnp.where(rvec < kcount, scpick, _NEG)
    out_ref[:, 6:8] = jnp.zeros((_OUT_ROWS, 2), jnp.float32)


def _nms(rows, cols):
    return pl.pallas_call(
        _nms_body,
        out_shape=jax.ShapeDtypeStruct((_OUT_ROWS, 8), jnp.float32),
        scratch_shapes=[
            pltpu.VMEM((_N, _N), jnp.bfloat16),
            pltpu.VMEM((_N, _N), jnp.bfloat16),
            pltpu.VMEM((8, _N), jnp.float32),
            pltpu.SMEM((1,), jnp.float32),
        ],
    )(rows, cols)


def kernel(scores, bbox_deltas, im_info, anchors):
    A = anchors.shape[0]
    H, W = scores.shape[2], scores.shape[3]
    fg = scores[0, A:].reshape(A, H * W)
    dl = bbox_deltas[0].reshape(A, 4, H * W)

    x1, y1, x2, y2, scm, key = _transform(fg, dl, anchors, im_info)

    def flat(v):
        return v.T.reshape(-1)

    npad = _NALL - A * H * W
    k2 = jnp.concatenate([flat(key), jnp.full((npad,), -1, jnp.int32)])
    tab = jnp.stack([flat(x1), flat(y1), flat(x2), flat(y2), flat(scm)], axis=1)
    tab = jnp.pad(tab, ((0, npad), (0, 11)))

    sorted_tab = _make_sc_topk()(k2, tab)
    cols = sorted_tab[:, :8]
    rows = cols.T

    out = _nms(rows, cols)
    return out[:POST_NMS_TOPN, :6]


# speedup vs baseline: 1.2415x; 1.1343x over previous
import functools

import jax
import jax.numpy as jnp
from jax import lax
from jax.experimental import pallas as pl
from jax.experimental.pallas import tpu as pltpu
from jax.experimental.pallas import tpu_sc as plsc

FEAT_STRIDE = 16.0
PRE_NMS_TOPN = 2000
POST_NMS_TOPN = 300
NMS_THRESH = 0.7
MIN_SIZE = 16.0

_N = 2048
_BLK = 128
_OUT_ROWS = 384
_NEG = -1e9


def _transform_body(fg_ref, dl_ref, anch_ref, im_ref, x1_ref, y1_ref, x2_ref, y2_ref, sc_ref, key_ref, t_ref):
    hw = jax.lax.broadcasted_iota(jnp.int32, (9, 2500), 1).astype(jnp.float32)
    row = jnp.floor((hw + 0.5) * (1.0 / 50.0))
    sy = row * FEAT_STRIDE
    sx = (hw - 50.0 * row) * FEAT_STRIDE

    ax1 = anch_ref[:, 0:1] + sx
    ay1 = anch_ref[:, 1:2] + sy
    ax2 = anch_ref[:, 2:3] + sx
    ay2 = anch_ref[:, 3:4] + sy

    widths = ax2 - ax1 + 1.0
    heights = ay2 - ay1 + 1.0
    ctr_x = ax1 + 0.5 * widths
    ctr_y = ay1 + 0.5 * heights

    dx = dl_ref[:, 0, :]
    dy = dl_ref[:, 1, :]
    dw = dl_ref[:, 2, :]
    dh = dl_ref[:, 3, :]

    pred_ctr_x = dx * widths + ctr_x
    pred_ctr_y = dy * heights + ctr_y
    pred_w = jnp.exp(dw) * widths
    pred_h = jnp.exp(dh) * heights

    im_h = im_ref[0:1, 0:1]
    im_w = im_ref[0:1, 1:2]
    im_scale = im_ref[0:1, 2:3]

    x1 = jnp.clip(pred_ctr_x - 0.5 * pred_w, 0.0, im_w - 1.0)
    y1 = jnp.clip(pred_ctr_y - 0.5 * pred_h, 0.0, im_h - 1.0)
    x2 = jnp.clip(pred_ctr_x + 0.5 * pred_w, 0.0, im_w - 1.0)
    y2 = jnp.clip(pred_ctr_y + 0.5 * pred_h, 0.0, im_h - 1.0)

    ws = x2 - x1 + 1.0
    hs = y2 - y1 + 1.0
    min_size = MIN_SIZE * im_scale
    valid = (ws >= min_size) & (hs >= min_size)

    x1_ref[...] = x1
    y1_ref[...] = y1
    x2_ref[...] = x2
    y2_ref[...] = y2
    scm = jnp.where(valid, fg_ref[...], _NEG)
    sc_ref[...] = scm
    b = jax.lax.bitcast_convert_type(scm, jnp.int32)
    u = jnp.where(b >= 0, b ^ jnp.int32(-2147483648), ~b)
    k2 = ~u
    key_ref[...] = k2

    kx = k2 ^ jnp.int32(-2147483648)

    def bit_round(i, p):
        cand = p | jax.lax.shift_left(jnp.int32(1), 31 - i)
        candx = cand ^ jnp.int32(-2147483648)
        cnt = jnp.sum((kx < candx).astype(jnp.int32))
        return jnp.where(cnt < PRE_NMS_TOPN, cand, p)

    t_val = jax.lax.fori_loop(0, 32, bit_round, jnp.int32(0))
    ic = jax.lax.broadcasted_iota(jnp.int32, (8, 128), 1)
    t_ref[...] = jnp.where(ic == 0, t_val, 0)


def _transform(fg, dl, anchors, im_info):
    out = jax.ShapeDtypeStruct((9, 2500), jnp.float32)
    outi = jax.ShapeDtypeStruct((9, 2500), jnp.int32)
    outt = jax.ShapeDtypeStruct((8, 128), jnp.int32)
    return pl.pallas_call(
        _transform_body,
        out_shape=(out, out, out, out, out, outi, outt),
    )(fg, dl, anchors, im_info)


_NALL = 22528
_NTILES = 16
_CHUNK = _NALL // _NTILES
_CVECS = _CHUNK // 16
_NSORT = 2048
_NCMP = 2304
_LPB = _NCMP // 16


def _sc_topk_body(k2_hbm, table_hbm, out_hbm,
                  lock2, hist, dig, gh, stmp, cnt16, pcl, lbufk, lbufn,
                  ka, na, kb, nb, scan4096, idx2d, rows, shist, pubcnt, skc,
                  snc, sem):
    cid = lax.axis_index("c")
    sid = lax.axis_index("s")
    lane = lax.broadcasted_iota(jnp.int32, (16,), 0)
    ones = jnp.ones((16,), jnp.int32)
    zeros16 = jnp.zeros((16,), jnp.int32)

    @pl.when(cid == 0)
    def _core0():
        base = sid * _CHUNK
        pltpu.sync_copy(k2_hbm.at[pl.ds(base, _CHUNK)], lock2)

        r = jnp.int32(PRE_NMS_TOPN)
        p = jnp.int32(0)
        for rnd in range(4):
            shift = 24 - 8 * rnd

            def zero_h(v, _):
                hist[pl.ds(v * 16, 16)] = zeros16
                return 0
            lax.fori_loop(0, 256, zero_h, 0)

            p_u = p.astype(jnp.uint32)

            def hist_body(v, _):
                kv = lock2[pl.ds(v * 16, 16)]
                ku = plsc.bitcast(kv, jnp.uint32)
                d = (lax.shift_right_logical(ku, jnp.uint32(shift))
                     & jnp.uint32(255)).astype(jnp.int32)
                if rnd == 0:
                    plsc.addupdate_scatter(hist, [lane * 256 + d], ones)
                else:
                    pm = lax.shift_right_logical(ku, jnp.uint32(shift + 8)) == p_u
                    plsc.addupdate_scatter(hist, [lane * 256 + d], ones, mask=pm)
                return 0
            lax.fori_loop(0, _CVECS, hist_body, 0)

            for c in range(16):
                acc = hist[pl.ds(c * 16, 16)]
                for l in range(1, 16):
                    acc = acc + hist[pl.ds(l * 256 + c * 16, 16)]
                dig[pl.ds(c * 16, 16)] = acc
            pltpu.sync_copy(dig, shist.at[rnd, sid])
            plsc.subcore_barrier()
            pltpu.sync_copy(shist.at[rnd], gh)

            running = jnp.int32(0)
            nless = jnp.int32(0)
            for c in range(16):
                s = gh[0, pl.ds(c * 16, 16)]
                for t in range(1, _NTILES):
                    s = s + gh[t, pl.ds(c * 16, 16)]
                cum = plsc.cumsum(s) + running
                stmp[pl.ds(c * 16, 16)] = cum
                running = running + jnp.sum(s)
                nless = nless + jnp.sum((cum < r).astype(jnp.int32))
            bstar = nless
            gidx = jnp.maximum(bstar - 1, 0) + zeros16
            sbm1 = jnp.max(plsc.load_gather(stmp, [gidx]))
            sbm1 = jnp.where(bstar > 0, sbm1, 0)
            r = r - sbm1
            p = (p << 8) | bstar
        tx = p ^ jnp.int32(-2147483648)
        tv = p
        r_eq = r

        def cnt_body(v, carry):
            ltc, eqc = carry
            kv = lock2[pl.ds(v * 16, 16)]
            lt = (kv ^ jnp.int32(-2147483648)) < tx
            eq = kv == tv
            return ltc + jnp.sum(lt.astype(jnp.int32)), eqc + jnp.sum(eq.astype(jnp.int32))
        lt_cnt, eq_cnt = lax.fori_loop(0, _CVECS, cnt_body,
                                       (jnp.int32(0), jnp.int32(0)))
        cnt16[...] = jnp.where(lane == 0, lt_cnt, jnp.where(lane == 1, eq_cnt, 0))
        pltpu.sync_copy(cnt16.at[pl.ds(0, 8)], pubcnt.at[sid])
        plsc.subcore_barrier()
        pltpu.sync_copy(pubcnt, pcl)
        lt_all = plsc.load_gather(pcl, [lane, zeros16])
        eq_all = plsc.load_gather(pcl, [lane, jnp.full((16,), 1, jnp.int32)])
        total_lt = jnp.sum(lt_all)
        take_total = jnp.int32(PRE_NMS_TOPN) - total_lt
        eq_excl = plsc.cumsum(eq_all) - eq_all
        take_all = jnp.clip(take_total - eq_excl, 0, eq_all)
        sel_all = lt_all + take_all
        padded_all = ((sel_all + 15) >> 4) << 4
        my_off = jnp.sum(jnp.where(lane < sid, padded_all, 0))
        my_take = jnp.sum(jnp.where(lane == sid, take_all, 0))
        my_pad = jnp.sum(jnp.where(lane == sid, padded_all, 0))
        total_padded = jnp.sum(padded_all)

        def comp_body(v, carry):
            cnt, eqc = carry
            kv = lock2[pl.ds(v * 16, 16)]
            lt = (kv ^ jnp.int32(-2147483648)) < tx
            eq = kv == tv
            eqi = eq.astype(jnp.int32)
            eqrank = eqc + plsc.cumsum(eqi) - 1
            take = lt | (eq & (eqrank < my_take))
            ti = take.astype(jnp.int32)
            posv = cnt + plsc.cumsum(ti) - 1
            plsc.store_scatter(lbufk, [posv], kv, mask=take)
            plsc.store_scatter(lbufn, [posv], base + v * 16 + lane, mask=take)
            return cnt + jnp.sum(ti), eqc + jnp.sum(eqi)
        my_sel, _ = lax.fori_loop(0, _CVECS, comp_body,
                                  (jnp.int32(0), jnp.int32(0)))
        padmask = lane < (my_pad - my_sel)
        plsc.store_scatter(lbufk, [my_sel + lane],
                           jnp.full((16,), -1, jnp.int32), mask=padmask)
        plsc.store_scatter(lbufn, [my_sel + lane],
                           jnp.full((16,), _NALL - 1, jnp.int32), mask=padmask)

        def dma_body(c, _):
            off = pl.multiple_of(my_off + c * 16, 16)

            @pl.when(c * 16 < my_pad)
            def _():
                pltpu.sync_copy(lbufk.at[pl.ds(c * 16, 16)],
                                skc.at[pl.ds(off, 16)])
                pltpu.sync_copy(lbufn.at[pl.ds(c * 16, 16)],
                                snc.at[pl.ds(off, 16)])
            return 0
        lax.fori_loop(0, _CVECS, dma_body, 0)
        plsc.subcore_barrier()

        @pl.when(sid == 0)
        def _tile0():
            pltpu.sync_copy(skc, ka)
            pltpu.sync_copy(snc, na)

            def tail_body(c, _):
                @pl.when(c * 16 >= total_padded)
                def _():
                    ka[pl.ds(c * 16, 16)] = jnp.full((16,), -1, jnp.int32)
                    na[pl.ds(c * 16, 16)] = jnp.full((16,), _NALL - 1, jnp.int32)
                return 0
            lax.fori_loop(0, _NCMP // 16, tail_body, 0)

            bufs = [(ka, na), (kb, nb)]
            for pno in range(4):
                shift = 8 * pno
                src_k, src_n = bufs[pno % 2]
                dst_k, dst_n = bufs[(pno + 1) % 2]

                def zero_h2(v, _):
                    hist[pl.ds(v * 16, 16)] = zeros16
                    return 0
                lax.fori_loop(0, 256, zero_h2, 0)

                def hist2(v, _):
                    kv = plsc.load_gather(src_k, [lane * _LPB + v])
                    ku = plsc.bitcast(kv, jnp.uint32)
                    d = (lax.shift_right_logical(ku, jnp.uint32(shift))
                         & jnp.uint32(255)).astype(jnp.int32)
                    plsc.addupdate_scatter(hist, [d * 16 + lane], ones)
                    return 0
                lax.fori_loop(0, _LPB, hist2, 0)

                def scan_b(c, running):
                    h = hist[pl.ds(c * 16, 16)]
                    scan4096[pl.ds(c * 16, 16)] = plsc.cumsum(h) - h + running
                    return running + jnp.sum(h)
                lax.fori_loop(0, 256, scan_b, jnp.int32(0))

                last = pno == 3

                def reorder(v, _):
                    eidx = lane * _LPB + v
                    kv = plsc.load_gather(src_k, [eidx])
                    nv = plsc.load_gather(src_n, [eidx])
                    ku = plsc.bitcast(kv, jnp.uint32)
                    d = (lax.shift_right_logical(ku, jnp.uint32(shift))
                         & jnp.uint32(255)).astype(jnp.int32)
                    hidx = d * 16 + lane
                    a = plsc.load_gather(scan4096, [hidx])
                    plsc.store_scatter(dst_k, [a], kv)
                    if last:
                        q = nv // 9
                        nv2 = (nv - q * 9) * 2500 + q
                    else:
                        nv2 = nv
                    plsc.store_scatter(dst_n, [a], nv2)
                    plsc.addupdate_scatter(scan4096, [hidx], ones)
                    return 0
                lax.fori_loop(0, _LPB, reorder, 0)

            for i in range(16):
                for u in range(8):
                    idx2d[i, pl.ds(u * 16, 16)] = na[pl.ds(i * 128 + u * 16, 16)]
            for i in range(16):
                pltpu.async_copy(table_hbm.at[idx2d.at[i]],
                                 rows.at[pl.ds(i * 128, 128)], sem).wait()
            zf = jnp.zeros((16,), jnp.float32)
            for j in range(PRE_NMS_TOPN, _NSORT):
                rows[j, :] = zf
            pltpu.sync_copy(rows, out_hbm)


@functools.cache
def _make_sc_topk():
    @functools.partial(
        pl.kernel,
        out_type=jax.ShapeDtypeStruct((_NSORT, 16), jnp.float32),
        mesh=plsc.VectorSubcoreMesh(core_axis_name="c", subcore_axis_name="s"),
        compiler_params=pltpu.CompilerParams(needs_layout_passes=False,
                                             use_tc_tiling_on_sc=False),
        scratch_types=[
            pltpu.VMEM((_CHUNK,), jnp.int32),
            pltpu.VMEM((4096,), jnp.int32),
            pltpu.VMEM((256,), jnp.int32),
            pltpu.VMEM((_NTILES, 256), jnp.int32),
            pltpu.VMEM((256,), jnp.int32),
            pltpu.VMEM((16,), jnp.int32),
            pltpu.VMEM((_NTILES, 8), jnp.int32),
            pltpu.VMEM((_CHUNK,), jnp.int32),
            pltpu.VMEM((_CHUNK,), jnp.int32),
            pltpu.VMEM((_NCMP,), jnp.int32),
            pltpu.VMEM((_NCMP,), jnp.int32),
            pltpu.VMEM((_NCMP,), jnp.int32),
            pltpu.VMEM((_NCMP,), jnp.int32),
            pltpu.VMEM((4096,), jnp.int32),
            pltpu.VMEM((16, 128), jnp.int32),
            pltpu.VMEM((_NSORT, 16), jnp.float32),
            pltpu.VMEM_SHARED((4, _NTILES, 256), jnp.int32),
            pltpu.VMEM_SHARED((_NTILES, 8), jnp.int32),
            pltpu.VMEM_SHARED((_NCMP,), jnp.int32),
            pltpu.VMEM_SHARED((_NCMP,), jnp.int32),
            pltpu.SemaphoreType.DMA,
        ],
    )
    def _sc_topk(k2_hbm, table_hbm, out_hbm, *scratch):
        _sc_topk_body(k2_hbm, table_hbm, out_hbm, *scratch)

    return _sc_topk


_P1 = 512


def _nms_block(rows4, cols_ref, b, ncols):
    x1r, y1r, x2r, y2r, area_r = rows4
    sl = pl.ds(b * _BLK, _BLK)
    x1c = cols_ref[sl, 0:1]
    y1c = cols_ref[sl, 1:2]
    x2c = cols_ref[sl, 2:3]
    y2c = cols_ref[sl, 3:4]
    area_c = (x2c - x1c + 1.0) * (y2c - y1c + 1.0)
    xx1 = jnp.maximum(x1c, x1r[:, :ncols])
    yy1 = jnp.maximum(y1c, y1r[:, :ncols])
    xx2 = jnp.minimum(x2c, x2r[:, :ncols])
    yy2 = jnp.minimum(y2c, y2r[:, :ncols])
    inter = jnp.maximum(xx2 - xx1 + 1.0, 0.0) * jnp.maximum(yy2 - yy1 + 1.0, 0.0)
    iou = inter / (area_c + area_r[:, :ncols] - inter)
    jg = b * _BLK + jax.lax.broadcasted_iota(jnp.int32, (_BLK, ncols), 0)
    ig = jax.lax.broadcasted_iota(jnp.int32, (_BLK, ncols), 1)
    sup = (iou > NMS_THRESH) & (jg < ig) & (ig < PRE_NMS_TOPN) & (jg < PRE_NMS_TOPN)
    return sup.astype(jnp.bfloat16), (jg <= ig).astype(jnp.bfloat16)


def _nms_body(rows_ref, cols_ref, out_ref, s_mat, lt_mat, pos_ref, kc_ref):
    x1r = rows_ref[0:1, :]
    y1r = rows_ref[1:2, :]
    x2r = rows_ref[2:3, :]
    y2r = rows_ref[3:4, :]
    area_r = (x2r - x1r + 1.0) * (y2r - y1r + 1.0)
    rows4 = (x1r, y1r, x2r, y2r, area_r)

    for b in range(_P1 // _BLK):
        s_blk, lt_blk = _nms_block(rows4, cols_ref, b, _P1)
        s_mat[pl.ds(b * _BLK, _BLK), pl.ds(0, _P1)] = s_blk
        lt_mat[pl.ds(b * _BLK, _BLK), pl.ds(0, _P1)] = lt_blk

    icol1 = jax.lax.broadcasted_iota(jnp.int32, (8, _P1), 1)
    keep0a = jnp.ones((8, _P1), jnp.float32)

    def cond_a(carry):
        _, changed, it = carry
        return changed & (it < _P1)

    def body_a(carry):
        keep, _, it = carry
        nk = keep
        for _ in range(4):
            sup = jnp.dot(nk.astype(jnp.bfloat16),
                          s_mat[pl.ds(0, _P1), pl.ds(0, _P1)],
                          preferred_element_type=jnp.float32)
            nk = jnp.where(sup < 0.5, 1.0, 0.0)
        changed = jnp.sum(jnp.abs(nk - keep)) > 0.0
        return nk, changed, it + 4

    keep_a, _, _ = jax.lax.while_loop(cond_a, body_a, (keep0a, True, 0))
    kept_a = jnp.sum(keep_a[0:1, :])

    @pl.when(kept_a >= float(POST_NMS_TOPN))
    def _fast():
        cumk = jnp.dot(keep_a.astype(jnp.bfloat16),
                       lt_mat[pl.ds(0, _P1), pl.ds(0, _P1)],
                       preferred_element_type=jnp.float32)
        pos_ref[:, 0:_P1] = jnp.where(keep_a > 0.5, cumk - 1.0, 1e9)
        kc_ref[0] = kept_a
        kc_ref[1] = 1.0

    @pl.when(kept_a < float(POST_NMS_TOPN))
    def _slow():
        for b in range(_N // _BLK):
            s_blk, lt_blk = _nms_block(rows4, cols_ref, b, _N)
            s_mat[pl.ds(b * _BLK, _BLK), :] = s_blk
            lt_mat[pl.ds(b * _BLK, _BLK), :] = lt_blk

        icol = jax.lax.broadcasted_iota(jnp.int32, (8, _N), 1)
        inb = (icol < PRE_NMS_TOPN).astype(jnp.float32)

        def cond(carry):
            _, changed, it = carry
            return changed & (it < _N)

        def body(carry):
            keep, _, it = carry
            sup = jnp.dot(keep.astype(jnp.bfloat16), s_mat[...],
                          preferred_element_type=jnp.float32)
            nk = jnp.where((sup < 0.5) & (icol < PRE_NMS_TOPN), 1.0, 0.0)
            changed = jnp.sum(jnp.abs(nk - keep)) > 0.0
            return nk, changed, it + 1

        keep, _, _ = jax.lax.while_loop(cond, body, (inb, True, 0))

        kcount = jnp.sum(keep[0:1, :])
        fill = (1.0 - keep) * inb
        cumk = jnp.dot(keep.astype(jnp.bfloat16), lt_mat[...],
                       preferred_element_type=jnp.float32)
        cumf = jnp.dot(fill.astype(jnp.bfloat16), lt_mat[...],
                       preferred_element_type=jnp.float32)
        pos = jnp.where(keep > 0.5, cumk - 1.0, kcount + cumf - 1.0)
        pos_ref[...] = jnp.where(inb > 0.5, pos, 1e9)
        kc_ref[0] = kcount
        kc_ref[1] = 0.0

    kcount = kc_ref[0]
    rvec = jax.lax.broadcasted_iota(jnp.int32, (_OUT_ROWS, 1), 0).astype(jnp.float32)

    def emit(ncols):
        pos1 = pos_ref[0:1, 0:ncols]
        riota = jax.lax.broadcasted_iota(
            jnp.int32, (_OUT_ROWS, ncols), 0).astype(jnp.float32)
        sel = riota == jnp.broadcast_to(pos1, (_OUT_ROWS, ncols))

        def pick(vals_row):
            v = jnp.broadcast_to(vals_row[:, 0:ncols], (_OUT_ROWS, ncols))
            return jnp.max(jnp.where(sel, v, -3.0e38), axis=1, keepdims=True)

        out_ref[:, 0:1] = jnp.zeros((_OUT_ROWS, 1), jnp.float32)
        out_ref[:, 1:2] = pick(x1r)
        out_ref[:, 2:3] = pick(y1r)
        out_ref[:, 3:4] = pick(x2r)
        out_ref[:, 4:5] = pick(y2r)
        scpick = pick(rows_ref[4:5, :])
        out_ref[:, 5:6] = jnp.where(rvec < kcount, scpick, _NEG)
        out_ref[:, 6:8] = jnp.zeros((_OUT_ROWS, 2), jnp.float32)

    fastf = kc_ref[1]

    @pl.when(fastf > 0.5)
    def _emit_fast():
        emit(_P1)

    @pl.when(fastf < 0.5)
    def _emit_slow():
        emit(_N)


def _nms(rows, cols):
    return pl.pallas_call(
        _nms_body,
        out_shape=jax.ShapeDtypeStruct((_OUT_ROWS, 8), jnp.float32),
        scratch_shapes=[
            pltpu.VMEM((_N, _N), jnp.bfloat16),
            pltpu.VMEM((_N, _N), jnp.bfloat16),
            pltpu.VMEM((8, _N), jnp.float32),
            pltpu.SMEM((2,), jnp.float32),
        ],
    )(rows, cols)


def kernel(scores, bbox_deltas, im_info, anchors):
    A = anchors.shape[0]
    H, W = scores.shape[2], scores.shape[3]
    fg = scores[0, A:].reshape(A, H * W)
    dl = bbox_deltas[0].reshape(A, 4, H * W)

    x1, y1, x2, y2, scm, key, tsel = _transform(fg, dl, anchors, im_info)

    def flat(v):
        return v.T.reshape(-1)

    npad = _NALL - A * H * W
    k2 = jnp.concatenate([flat(key), jnp.full((npad,), -1, jnp.int32)])
    tab = jnp.stack([x1, y1, x2, y2, scm], axis=-1)
    tab = jnp.pad(tab, ((0, 0), (0, 0), (0, 11))).reshape(A * H * W, 16)
    tab = jnp.pad(tab, ((0, npad), (0, 0)))

    sorted_tab = _make_sc_topk()(k2, tab)
    cols = sorted_tab[:, :8]
    rows = cols.T

    out = _nms(rows, cols)
    return out[:POST_NMS_TOPN, :6]


# speedup vs baseline: 1.3661x; 1.1003x over previous
import functools

import jax
import jax.numpy as jnp
from jax import lax
from jax.experimental import pallas as pl
from jax.experimental.pallas import tpu as pltpu
from jax.experimental.pallas import tpu_sc as plsc

FEAT_STRIDE = 16.0
PRE_NMS_TOPN = 2000
POST_NMS_TOPN = 300
NMS_THRESH = 0.7
MIN_SIZE = 16.0

_N = 2048
_BLK = 128
_OUT_ROWS = 384
_NEG = -1e9


def _transform_body(fg_ref, dl_ref, anch_ref, im_ref, x1_ref, y1_ref, x2_ref, y2_ref, sc_ref, key_ref, t_ref):
    hw = jax.lax.broadcasted_iota(jnp.int32, (9, 2500), 1).astype(jnp.float32)
    row = jnp.floor((hw + 0.5) * (1.0 / 50.0))
    sy = row * FEAT_STRIDE
    sx = (hw - 50.0 * row) * FEAT_STRIDE

    ax1 = anch_ref[:, 0:1] + sx
    ay1 = anch_ref[:, 1:2] + sy
    ax2 = anch_ref[:, 2:3] + sx
    ay2 = anch_ref[:, 3:4] + sy

    widths = ax2 - ax1 + 1.0
    heights = ay2 - ay1 + 1.0
    ctr_x = ax1 + 0.5 * widths
    ctr_y = ay1 + 0.5 * heights

    dx = dl_ref[:, 0, :]
    dy = dl_ref[:, 1, :]
    dw = dl_ref[:, 2, :]
    dh = dl_ref[:, 3, :]

    pred_ctr_x = dx * widths + ctr_x
    pred_ctr_y = dy * heights + ctr_y
    pred_w = jnp.exp(dw) * widths
    pred_h = jnp.exp(dh) * heights

    im_h = im_ref[0:1, 0:1]
    im_w = im_ref[0:1, 1:2]
    im_scale = im_ref[0:1, 2:3]

    x1 = jnp.clip(pred_ctr_x - 0.5 * pred_w, 0.0, im_w - 1.0)
    y1 = jnp.clip(pred_ctr_y - 0.5 * pred_h, 0.0, im_h - 1.0)
    x2 = jnp.clip(pred_ctr_x + 0.5 * pred_w, 0.0, im_w - 1.0)
    y2 = jnp.clip(pred_ctr_y + 0.5 * pred_h, 0.0, im_h - 1.0)

    ws = x2 - x1 + 1.0
    hs = y2 - y1 + 1.0
    min_size = MIN_SIZE * im_scale
    valid = (ws >= min_size) & (hs >= min_size)

    x1_ref[...] = x1
    y1_ref[...] = y1
    x2_ref[...] = x2
    y2_ref[...] = y2
    scm = jnp.where(valid, fg_ref[...], _NEG)
    sc_ref[...] = scm
    b = jax.lax.bitcast_convert_type(scm, jnp.int32)
    u = jnp.where(b >= 0, b ^ jnp.int32(-2147483648), ~b)
    k2 = ~u
    key_ref[...] = k2

    kx = k2 ^ jnp.int32(-2147483648)

    def bit_round(i, p):
        cand = p | jax.lax.shift_left(jnp.int32(1), 31 - i)
        candx = cand ^ jnp.int32(-2147483648)
        cnt = jnp.sum((kx < candx).astype(jnp.int32))
        return jnp.where(cnt < PRE_NMS_TOPN, cand, p)

    t_val = jax.lax.fori_loop(0, 32, bit_round, jnp.int32(0))
    t_ref[...] = jnp.zeros((8, 128), jnp.int32) + t_val


def _transform(fg, dl, anchors, im_info):
    out = jax.ShapeDtypeStruct((9, 2500), jnp.float32)
    outi = jax.ShapeDtypeStruct((9, 2500), jnp.int32)
    outt = jax.ShapeDtypeStruct((8, 128), jnp.int32)
    return pl.pallas_call(
        _transform_body,
        out_shape=(out, out, out, out, out, outi, outt),
    )(fg, dl, anchors, im_info)


_NALL = 22528
_NTILES = 16
_CHUNK = _NALL // _NTILES
_CVECS = _CHUNK // 16
_NSORT = 2048
_NCMP = 2304
_LPB = _NCMP // 16


def _sc_topk_body(k2_hbm, t_hbm, table_hbm, out_hbm,
                  lock2, hist, cnt16, pcl, lbufk, lbufn,
                  ka, na, kb, nb, scan4096, idx2d, rows, pubcnt, skc,
                  snc, sem):
    cid = lax.axis_index("c")
    sid = lax.axis_index("s")
    lane = lax.broadcasted_iota(jnp.int32, (16,), 0)
    ones = jnp.ones((16,), jnp.int32)
    zeros16 = jnp.zeros((16,), jnp.int32)

    @pl.when(cid == 0)
    def _core0():
        base = sid * _CHUNK
        pltpu.sync_copy(k2_hbm.at[pl.ds(base, _CHUNK)], lock2)
        pltpu.sync_copy(t_hbm.at[0, pl.ds(0, 16)], cnt16)
        tv = jnp.max(cnt16[...])
        tx = tv ^ jnp.int32(-2147483648)

        def cnt_body(v, carry):
            ltc, eqc = carry
            kv = lock2[pl.ds(v * 16, 16)]
            lt = (kv ^ jnp.int32(-2147483648)) < tx
            eq = kv == tv
            return ltc + jnp.sum(lt.astype(jnp.int32)), eqc + jnp.sum(eq.astype(jnp.int32))
        lt_cnt, eq_cnt = lax.fori_loop(0, _CVECS, cnt_body,
                                       (jnp.int32(0), jnp.int32(0)))
        cnt16[...] = jnp.where(lane == 0, lt_cnt, jnp.where(lane == 1, eq_cnt, 0))
        pltpu.sync_copy(cnt16.at[pl.ds(0, 8)], pubcnt.at[sid])
        plsc.subcore_barrier()
        pltpu.sync_copy(pubcnt, pcl)
        lt_all = plsc.load_gather(pcl, [lane, zeros16])
        eq_all = plsc.load_gather(pcl, [lane, jnp.full((16,), 1, jnp.int32)])
        total_lt = jnp.sum(lt_all)
        take_total = jnp.int32(PRE_NMS_TOPN) - total_lt
        eq_excl = plsc.cumsum(eq_all) - eq_all
        take_all = jnp.clip(take_total - eq_excl, 0, eq_all)
        sel_all = lt_all + take_all
        padded_all = ((sel_all + 15) >> 4) << 4
        my_off = jnp.sum(jnp.where(lane < sid, padded_all, 0))
        my_take = jnp.sum(jnp.where(lane == sid, take_all, 0))
        my_pad = jnp.sum(jnp.where(lane == sid, padded_all, 0))
        total_padded = jnp.sum(padded_all)

        def comp_body(v, carry):
            cnt, eqc = carry
            kv = lock2[pl.ds(v * 16, 16)]
            lt = (kv ^ jnp.int32(-2147483648)) < tx
            eq = kv == tv
            eqi = eq.astype(jnp.int32)
            eqrank = eqc + plsc.cumsum(eqi) - 1
            take = lt | (eq & (eqrank < my_take))
            ti = take.astype(jnp.int32)
            posv = cnt + plsc.cumsum(ti) - 1
            plsc.store_scatter(lbufk, [posv], kv, mask=take)
            plsc.store_scatter(lbufn, [posv], base + v * 16 + lane, mask=take)
            return cnt + jnp.sum(ti), eqc + jnp.sum(eqi)
        my_sel, _ = lax.fori_loop(0, _CVECS, comp_body,
                                  (jnp.int32(0), jnp.int32(0)))
        padmask = lane < (my_pad - my_sel)
        plsc.store_scatter(lbufk, [my_sel + lane],
                           jnp.full((16,), -1, jnp.int32), mask=padmask)
        plsc.store_scatter(lbufn, [my_sel + lane],
                           jnp.full((16,), _NALL - 1, jnp.int32), mask=padmask)

        def dma_body(c, _):
            off = pl.multiple_of(my_off + c * 16, 16)

            @pl.when(c * 16 < my_pad)
            def _():
                pltpu.sync_copy(lbufk.at[pl.ds(c * 16, 16)],
                                skc.at[pl.ds(off, 16)])
                pltpu.sync_copy(lbufn.at[pl.ds(c * 16, 16)],
                                snc.at[pl.ds(off, 16)])
            return 0
        lax.fori_loop(0, _CVECS, dma_body, 0)
        plsc.subcore_barrier()

        @pl.when(sid == 0)
        def _tile0():
            pltpu.sync_copy(skc, ka)
            pltpu.sync_copy(snc, na)

            def tail_body(c, _):
                @pl.when(c * 16 >= total_padded)
                def _():
                    ka[pl.ds(c * 16, 16)] = jnp.full((16,), -1, jnp.int32)
                    na[pl.ds(c * 16, 16)] = jnp.full((16,), _NALL - 1, jnp.int32)
                return 0
            lax.fori_loop(0, _NCMP // 16, tail_body, 0)

            bufs = [(ka, na), (kb, nb)]
            for pno in range(4):
                shift = 8 * pno
                src_k, src_n = bufs[pno % 2]
                dst_k, dst_n = bufs[(pno + 1) % 2]

                def zero_h2(v, _):
                    hist[pl.ds(v * 16, 16)] = zeros16
                    return 0
                lax.fori_loop(0, 256, zero_h2, 0)

                def hist2(v, _):
                    kv = plsc.load_gather(src_k, [lane * _LPB + v])
                    ku = plsc.bitcast(kv, jnp.uint32)
                    d = (lax.shift_right_logical(ku, jnp.uint32(shift))
                         & jnp.uint32(255)).astype(jnp.int32)
                    plsc.addupdate_scatter(hist, [d * 16 + lane], ones)
                    return 0
                lax.fori_loop(0, _LPB, hist2, 0)

                def scan_b(c, running):
                    h = hist[pl.ds(c * 16, 16)]
                    scan4096[pl.ds(c * 16, 16)] = plsc.cumsum(h) - h + running
                    return running + jnp.sum(h)
                lax.fori_loop(0, 256, scan_b, jnp.int32(0))

                last = pno == 3

                def reorder(v, _):
                    eidx = lane * _LPB + v
                    kv = plsc.load_gather(src_k, [eidx])
                    nv = plsc.load_gather(src_n, [eidx])
                    ku = plsc.bitcast(kv, jnp.uint32)
                    d = (lax.shift_right_logical(ku, jnp.uint32(shift))
                         & jnp.uint32(255)).astype(jnp.int32)
                    hidx = d * 16 + lane
                    a = plsc.load_gather(scan4096, [hidx])
                    plsc.store_scatter(dst_k, [a], kv)
                    if last:
                        q = nv // 9
                        nv2 = (nv - q * 9) * 2500 + q
                    else:
                        nv2 = nv
                    plsc.store_scatter(dst_n, [a], nv2)
                    plsc.addupdate_scatter(scan4096, [hidx], ones)
                    return 0
                lax.fori_loop(0, _LPB, reorder, 0)

            for i in range(16):
                for u in range(8):
                    idx2d[i, pl.ds(u * 16, 16)] = na[pl.ds(i * 128 + u * 16, 16)]
            for i in range(16):
                pltpu.async_copy(table_hbm.at[idx2d.at[i]],
                                 rows.at[pl.ds(i * 128, 128)], sem).wait()
            zf = jnp.zeros((16,), jnp.float32)
            for j in range(PRE_NMS_TOPN, _NSORT):
                rows[j, :] = zf
            pltpu.sync_copy(rows, out_hbm)


@functools.cache
def _make_sc_topk():
    @functools.partial(
        pl.kernel,
        out_type=jax.ShapeDtypeStruct((_NSORT, 16), jnp.float32),
        mesh=plsc.VectorSubcoreMesh(core_axis_name="c", subcore_axis_name="s"),
        compiler_params=pltpu.CompilerParams(needs_layout_passes=False,
                                             use_tc_tiling_on_sc=False),
        scratch_types=[
            pltpu.VMEM((_CHUNK,), jnp.int32),
            pltpu.VMEM((4096,), jnp.int32),
            pltpu.VMEM((16,), jnp.int32),
            pltpu.VMEM((_NTILES, 8), jnp.int32),
            pltpu.VMEM((_CHUNK,), jnp.int32),
            pltpu.VMEM((_CHUNK,), jnp.int32),
            pltpu.VMEM((_NCMP,), jnp.int32),
            pltpu.VMEM((_NCMP,), jnp.int32),
            pltpu.VMEM((_NCMP,), jnp.int32),
            pltpu.VMEM((_NCMP,), jnp.int32),
            pltpu.VMEM((4096,), jnp.int32),
            pltpu.VMEM((16, 128), jnp.int32),
            pltpu.VMEM((_NSORT, 16), jnp.float32),
            pltpu.VMEM_SHARED((_NTILES, 8), jnp.int32),
            pltpu.VMEM_SHARED((_NCMP,), jnp.int32),
            pltpu.VMEM_SHARED((_NCMP,), jnp.int32),
            pltpu.SemaphoreType.DMA,
        ],
    )
    def _sc_topk(k2_hbm, t_hbm, table_hbm, out_hbm, *scratch):
        _sc_topk_body(k2_hbm, t_hbm, table_hbm, out_hbm, *scratch)

    return _sc_topk


_P1 = 512


def _nms_block(rows4, cols_ref, b, ncols):
    x1r, y1r, x2r, y2r, area_r = rows4
    sl = pl.ds(b * _BLK, _BLK)
    x1c = cols_ref[sl, 0:1]
    y1c = cols_ref[sl, 1:2]
    x2c = cols_ref[sl, 2:3]
    y2c = cols_ref[sl, 3:4]
    area_c = (x2c - x1c + 1.0) * (y2c - y1c + 1.0)
    xx1 = jnp.maximum(x1c, x1r[:, :ncols])
    yy1 = jnp.maximum(y1c, y1r[:, :ncols])
    xx2 = jnp.minimum(x2c, x2r[:, :ncols])
    yy2 = jnp.minimum(y2c, y2r[:, :ncols])
    inter = jnp.maximum(xx2 - xx1 + 1.0, 0.0) * jnp.maximum(yy2 - yy1 + 1.0, 0.0)
    iou = inter / (area_c + area_r[:, :ncols] - inter)
    jg = b * _BLK + jax.lax.broadcasted_iota(jnp.int32, (_BLK, ncols), 0)
    ig = jax.lax.broadcasted_iota(jnp.int32, (_BLK, ncols), 1)
    sup = (iou > NMS_THRESH) & (jg < ig) & (ig < PRE_NMS_TOPN) & (jg < PRE_NMS_TOPN)
    return sup.astype(jnp.bfloat16), (jg <= ig).astype(jnp.bfloat16)


def _nms_body(rows_ref, cols_ref, out_ref, s_mat, lt_mat, pos_ref, kc_ref):
    x1r = rows_ref[0:1, :]
    y1r = rows_ref[1:2, :]
    x2r = rows_ref[2:3, :]
    y2r = rows_ref[3:4, :]
    area_r = (x2r - x1r + 1.0) * (y2r - y1r + 1.0)
    rows4 = (x1r, y1r, x2r, y2r, area_r)

    for b in range(_P1 // _BLK):
        s_blk, lt_blk = _nms_block(rows4, cols_ref, b, _P1)
        s_mat[pl.ds(b * _BLK, _BLK), pl.ds(0, _P1)] = s_blk
        lt_mat[pl.ds(b * _BLK, _BLK), pl.ds(0, _P1)] = lt_blk

    icol1 = jax.lax.broadcasted_iota(jnp.int32, (8, _P1), 1)
    keep0a = jnp.ones((8, _P1), jnp.float32)

    def cond_a(carry):
        _, changed, it = carry
        return changed & (it < _P1)

    def body_a(carry):
        keep, _, it = carry
        nk = keep
        for _ in range(4):
            sup = jnp.dot(nk.astype(jnp.bfloat16),
                          s_mat[pl.ds(0, _P1), pl.ds(0, _P1)],
                          preferred_element_type=jnp.float32)
            nk = jnp.where(sup < 0.5, 1.0, 0.0)
        changed = jnp.sum(jnp.abs(nk - keep)) > 0.0
        return nk, changed, it + 4

    keep_a, _, _ = jax.lax.while_loop(cond_a, body_a, (keep0a, True, 0))
    kept_a = jnp.sum(keep_a[0:1, :])

    @pl.when(kept_a >= float(POST_NMS_TOPN))
    def _fast():
        cumk = jnp.dot(keep_a.astype(jnp.bfloat16),
                       lt_mat[pl.ds(0, _P1), pl.ds(0, _P1)],
                       preferred_element_type=jnp.float32)
        pos_ref[:, 0:_P1] = jnp.where(keep_a > 0.5, cumk - 1.0, 1e9)
        kc_ref[0] = kept_a
        kc_ref[1] = 1.0

    @pl.when(kept_a < float(POST_NMS_TOPN))
    def _slow():
        for b in range(_N // _BLK):
            s_blk, lt_blk = _nms_block(rows4, cols_ref, b, _N)
            s_mat[pl.ds(b * _BLK, _BLK), :] = s_blk
            lt_mat[pl.ds(b * _BLK, _BLK), :] = lt_blk

        icol = jax.lax.broadcasted_iota(jnp.int32, (8, _N), 1)
        inb = (icol < PRE_NMS_TOPN).astype(jnp.float32)

        def cond(carry):
            _, changed, it = carry
            return changed & (it < _N)

        def body(carry):
            keep, _, it = carry
            sup = jnp.dot(keep.astype(jnp.bfloat16), s_mat[...],
                          preferred_element_type=jnp.float32)
            nk = jnp.where((sup < 0.5) & (icol < PRE_NMS_TOPN), 1.0, 0.0)
            changed = jnp.sum(jnp.abs(nk - keep)) > 0.0
            return nk, changed, it + 1

        keep, _, _ = jax.lax.while_loop(cond, body, (inb, True, 0))

        kcount = jnp.sum(keep[0:1, :])
        fill = (1.0 - keep) * inb
        cumk = jnp.dot(keep.astype(jnp.bfloat16), lt_mat[...],
                       preferred_element_type=jnp.float32)
        cumf = jnp.dot(fill.astype(jnp.bfloat16), lt_mat[...],
                       preferred_element_type=jnp.float32)
        pos = jnp.where(keep > 0.5, cumk - 1.0, kcount + cumf - 1.0)
        pos_ref[...] = jnp.where(inb > 0.5, pos, 1e9)
        kc_ref[0] = kcount
        kc_ref[1] = 0.0

    kcount = kc_ref[0]
    rvec = jax.lax.broadcasted_iota(jnp.int32, (_OUT_ROWS, 1), 0).astype(jnp.float32)

    def emit(ncols):
        pos1 = pos_ref[0:1, 0:ncols]
        riota = jax.lax.broadcasted_iota(
            jnp.int32, (_OUT_ROWS, ncols), 0).astype(jnp.float32)
        sel = riota == jnp.broadcast_to(pos1, (_OUT_ROWS, ncols))

        def pick(vals_row):
            v = jnp.broadcast_to(vals_row[:, 0:ncols], (_OUT_ROWS, ncols))
            return jnp.max(jnp.where(sel, v, -3.0e38), axis=1, keepdims=True)

        out_ref[:, 0:1] = jnp.zeros((_OUT_ROWS, 1), jnp.float32)
        out_ref[:, 1:2] = pick(x1r)
        out_ref[:, 2:3] = pick(y1r)
        out_ref[:, 3:4] = pick(x2r)
        out_ref[:, 4:5] = pick(y2r)
        scpick = pick(rows_ref[4:5, :])
        out_ref[:, 5:6] = jnp.where(rvec < kcount, scpick, _NEG)
        out_ref[:, 6:8] = jnp.zeros((_OUT_ROWS, 2), jnp.float32)

    fastf = kc_ref[1]

    @pl.when(fastf > 0.5)
    def _emit_fast():
        emit(_P1)

    @pl.when(fastf < 0.5)
    def _emit_slow():
        emit(_N)


def _nms(rows, cols):
    return pl.pallas_call(
        _nms_body,
        out_shape=jax.ShapeDtypeStruct((_OUT_ROWS, 8), jnp.float32),
        scratch_shapes=[
            pltpu.VMEM((_N, _N), jnp.bfloat16),
            pltpu.VMEM((_N, _N), jnp.bfloat16),
            pltpu.VMEM((8, _N), jnp.float32),
            pltpu.SMEM((2,), jnp.float32),
        ],
    )(rows, cols)


def kernel(scores, bbox_deltas, im_info, anchors):
    A = anchors.shape[0]
    H, W = scores.shape[2], scores.shape[3]
    fg = scores[0, A:].reshape(A, H * W)
    dl = bbox_deltas[0].reshape(A, 4, H * W)

    x1, y1, x2, y2, scm, key, tsel = _transform(fg, dl, anchors, im_info)

    def flat(v):
        return v.T.reshape(-1)

    npad = _NALL - A * H * W
    k2 = jnp.concatenate([flat(key), jnp.full((npad,), -1, jnp.int32)])
    tab = jnp.stack([x1, y1, x2, y2, scm], axis=-1)
    tab = jnp.pad(tab, ((0, 0), (0, 0), (0, 11))).reshape(A * H * W, 16)
    tab = jnp.pad(tab, ((0, npad), (0, 0)))

    sorted_tab = _make_sc_topk()(k2, tsel, tab)
    cols = sorted_tab[:, :8]
    rows = cols.T

    out = _nms(rows, cols)
    return out[:POST_NMS_TOPN, :6]


# speedup vs baseline: 1.4648x; 1.0723x over previous
import functools

import jax
import jax.numpy as jnp
from jax import lax
from jax.experimental import pallas as pl
from jax.experimental.pallas import tpu as pltpu
from jax.experimental.pallas import tpu_sc as plsc

FEAT_STRIDE = 16.0
PRE_NMS_TOPN = 2000
POST_NMS_TOPN = 300
NMS_THRESH = 0.7
MIN_SIZE = 16.0

_N = 2048
_BLK = 128
_OUT_ROWS = 384
_NEG = -1e9


def _transform_body(fg_ref, dl_ref, anch_ref, im_ref, x1_ref, y1_ref, x2_ref, y2_ref, sc_ref, key_ref, t_ref):
    hw = jax.lax.broadcasted_iota(jnp.int32, (9, 2500), 1).astype(jnp.float32)
    row = jnp.floor((hw + 0.5) * (1.0 / 50.0))
    sy = row * FEAT_STRIDE
    sx = (hw - 50.0 * row) * FEAT_STRIDE

    ax1 = anch_ref[:, 0:1] + sx
    ay1 = anch_ref[:, 1:2] + sy
    ax2 = anch_ref[:, 2:3] + sx
    ay2 = anch_ref[:, 3:4] + sy

    widths = ax2 - ax1 + 1.0
    heights = ay2 - ay1 + 1.0
    ctr_x = ax1 + 0.5 * widths
    ctr_y = ay1 + 0.5 * heights

    dx = dl_ref[:, 0, :]
    dy = dl_ref[:, 1, :]
    dw = dl_ref[:, 2, :]
    dh = dl_ref[:, 3, :]

    pred_ctr_x = dx * widths + ctr_x
    pred_ctr_y = dy * heights + ctr_y
    pred_w = jnp.exp(dw) * widths
    pred_h = jnp.exp(dh) * heights

    im_h = im_ref[0:1, 0:1]
    im_w = im_ref[0:1, 1:2]
    im_scale = im_ref[0:1, 2:3]

    x1 = jnp.clip(pred_ctr_x - 0.5 * pred_w, 0.0, im_w - 1.0)
    y1 = jnp.clip(pred_ctr_y - 0.5 * pred_h, 0.0, im_h - 1.0)
    x2 = jnp.clip(pred_ctr_x + 0.5 * pred_w, 0.0, im_w - 1.0)
    y2 = jnp.clip(pred_ctr_y + 0.5 * pred_h, 0.0, im_h - 1.0)

    ws = x2 - x1 + 1.0
    hs = y2 - y1 + 1.0
    min_size = MIN_SIZE * im_scale
    valid = (ws >= min_size) & (hs >= min_size)

    x1_ref[...] = x1
    y1_ref[...] = y1
    x2_ref[...] = x2
    y2_ref[...] = y2
    scm = jnp.where(valid, fg_ref[...], _NEG)
    sc_ref[...] = scm
    b = jax.lax.bitcast_convert_type(scm, jnp.int32)
    u = jnp.where(b >= 0, b ^ jnp.int32(-2147483648), ~b)
    k2 = ~u
    key_ref[...] = k2

    kx = k2 ^ jnp.int32(-2147483648)

    def bit_round(i, p):
        cand = p | jax.lax.shift_left(jnp.int32(1), 31 - i)
        candx = cand ^ jnp.int32(-2147483648)
        cnt = jnp.sum((kx < candx).astype(jnp.int32))
        return jnp.where(cnt < PRE_NMS_TOPN, cand, p)

    t_val = jax.lax.fori_loop(0, 32, bit_round, jnp.int32(0))
    t_ref[...] = jnp.zeros((8, 128), jnp.int32) + t_val


def _transform(fg, dl, anchors, im_info):
    out = jax.ShapeDtypeStruct((9, 2500), jnp.float32)
    outi = jax.ShapeDtypeStruct((9, 2500), jnp.int32)
    outt = jax.ShapeDtypeStruct((8, 128), jnp.int32)
    return pl.pallas_call(
        _transform_body,
        out_shape=(out, out, out, out, out, outi, outt),
    )(fg, dl, anchors, im_info)


_NALL = 22528
_NTILES = 16
_CHUNK = _NALL // _NTILES
_CVECS = _CHUNK // 16
_NSORT = 2048
_NCMP = 2304
_LPB = _NCMP // 16


def _sc_topk_body(k2_hbm, t_hbm, table_hbm, out_hbm,
                  lock2, hist, cnt16, pcl, lbufk, lbufn,
                  ka, na, kb, nb, scan4096, idx2d, rows, pubcnt, skc,
                  snc, sem):
    cid = lax.axis_index("c")
    sid = lax.axis_index("s")
    lane = lax.broadcasted_iota(jnp.int32, (16,), 0)
    ones = jnp.ones((16,), jnp.int32)
    zeros16 = jnp.zeros((16,), jnp.int32)

    @pl.when(cid == 0)
    def _core0():
        base = sid * _CHUNK
        pltpu.sync_copy(k2_hbm.at[pl.ds(base, _CHUNK)], lock2)
        pltpu.sync_copy(t_hbm.at[0, pl.ds(0, 16)], cnt16)
        tv = jnp.max(cnt16[...])
        tx = tv ^ jnp.int32(-2147483648)

        def cnt_body(v, carry):
            ltc, eqc = carry
            kv = lock2[pl.ds(v * 16, 16)]
            lt = (kv ^ jnp.int32(-2147483648)) < tx
            eq = kv == tv
            return ltc + jnp.sum(lt.astype(jnp.int32)), eqc + jnp.sum(eq.astype(jnp.int32))
        lt_cnt, eq_cnt = lax.fori_loop(0, _CVECS, cnt_body,
                                       (jnp.int32(0), jnp.int32(0)))
        cnt16[...] = jnp.where(lane == 0, lt_cnt, jnp.where(lane == 1, eq_cnt, 0))
        pltpu.sync_copy(cnt16.at[pl.ds(0, 8)], pubcnt.at[sid])
        plsc.subcore_barrier()
        pltpu.sync_copy(pubcnt, pcl)
        lt_all = plsc.load_gather(pcl, [lane, zeros16])
        eq_all = plsc.load_gather(pcl, [lane, jnp.full((16,), 1, jnp.int32)])
        total_lt = jnp.sum(lt_all)
        take_total = jnp.int32(PRE_NMS_TOPN) - total_lt
        eq_excl = plsc.cumsum(eq_all) - eq_all
        take_all = jnp.clip(take_total - eq_excl, 0, eq_all)
        sel_all = lt_all + take_all
        padded_all = ((sel_all + 15) >> 4) << 4
        my_off = jnp.sum(jnp.where(lane < sid, padded_all, 0))
        my_take = jnp.sum(jnp.where(lane == sid, take_all, 0))
        my_pad = jnp.sum(jnp.where(lane == sid, padded_all, 0))
        total_padded = jnp.sum(padded_all)

        def comp_body(v, carry):
            cnt, eqc = carry
            kv = lock2[pl.ds(v * 16, 16)]
            lt = (kv ^ jnp.int32(-2147483648)) < tx
            eq = kv == tv
            eqi = eq.astype(jnp.int32)
            eqrank = eqc + plsc.cumsum(eqi) - 1
            take = lt | (eq & (eqrank < my_take))
            ti = take.astype(jnp.int32)
            posv = cnt + plsc.cumsum(ti) - 1
            plsc.store_scatter(lbufk, [posv], kv, mask=take)
            plsc.store_scatter(lbufn, [posv], base + v * 16 + lane, mask=take)
            return cnt + jnp.sum(ti), eqc + jnp.sum(eqi)
        my_sel, _ = lax.fori_loop(0, _CVECS, comp_body,
                                  (jnp.int32(0), jnp.int32(0)))
        padmask = lane < (my_pad - my_sel)
        plsc.store_scatter(lbufk, [my_sel + lane],
                           jnp.full((16,), -1, jnp.int32), mask=padmask)
        plsc.store_scatter(lbufn, [my_sel + lane],
                           jnp.full((16,), _NALL - 1, jnp.int32), mask=padmask)

        def dma_body(c, _):
            off = pl.multiple_of(my_off + c * 16, 16)

            @pl.when(c * 16 < my_pad)
            def _():
                pltpu.sync_copy(lbufk.at[pl.ds(c * 16, 16)],
                                skc.at[pl.ds(off, 16)])
                pltpu.sync_copy(lbufn.at[pl.ds(c * 16, 16)],
                                snc.at[pl.ds(off, 16)])
            return 0
        lax.fori_loop(0, _CVECS, dma_body, 0)
        plsc.subcore_barrier()

        @pl.when(sid == 0)
        def _tile0():
            pltpu.sync_copy(skc, ka)
            pltpu.sync_copy(snc, na)

            def tail_body(c, _):
                @pl.when(c * 16 >= total_padded)
                def _():
                    ka[pl.ds(c * 16, 16)] = jnp.full((16,), -1, jnp.int32)
                    na[pl.ds(c * 16, 16)] = jnp.full((16,), _NALL - 1, jnp.int32)
                return 0
            lax.fori_loop(0, _NCMP // 16, tail_body, 0)

            bufs = [(ka, na), (kb, nb)]
            for pno in range(4):
                shift = 8 * pno
                src_k, src_n = bufs[pno % 2]
                dst_k, dst_n = bufs[(pno + 1) % 2]

                def zero_h2(v, _):
                    hist[pl.ds(v * 16, 16)] = zeros16
                    return 0
                lax.fori_loop(0, 256, zero_h2, 0)

                def hist2(v, _):
                    kv = plsc.load_gather(src_k, [lane * _LPB + v])
                    ku = plsc.bitcast(kv, jnp.uint32)
                    d = (lax.shift_right_logical(ku, jnp.uint32(shift))
                         & jnp.uint32(255)).astype(jnp.int32)
                    plsc.addupdate_scatter(hist, [d * 16 + lane], ones)
                    return 0
                lax.fori_loop(0, _LPB, hist2, 0)

                def scan_b(c, running):
                    h = hist[pl.ds(c * 16, 16)]
                    scan4096[pl.ds(c * 16, 16)] = plsc.cumsum(h) - h + running
                    return running + jnp.sum(h)
                lax.fori_loop(0, 256, scan_b, jnp.int32(0))

                last = pno == 3

                def reorder(v, _):
                    eidx = lane * _LPB + v
                    kv = plsc.load_gather(src_k, [eidx])
                    nv = plsc.load_gather(src_n, [eidx])
                    ku = plsc.bitcast(kv, jnp.uint32)
                    d = (lax.shift_right_logical(ku, jnp.uint32(shift))
                         & jnp.uint32(255)).astype(jnp.int32)
                    hidx = d * 16 + lane
                    a = plsc.load_gather(scan4096, [hidx])
                    plsc.store_scatter(dst_k, [a], kv)
                    if last:
                        q = nv // 9
                        nv2 = (nv - q * 9) * 2500 + q
                    else:
                        nv2 = nv
                    plsc.store_scatter(dst_n, [a], nv2)
                    plsc.addupdate_scatter(scan4096, [hidx], ones)
                    return 0
                lax.fori_loop(0, _LPB, reorder, 0)

            for i in range(16):
                for u in range(8):
                    idx2d[i, pl.ds(u * 16, 16)] = na[pl.ds(i * 128 + u * 16, 16)]
            copies = [
                pltpu.async_copy(table_hbm.at[idx2d.at[i]],
                                 rows.at[pl.ds(i * 128, 128)], sem)
                for i in range(16)
            ]
            for c in copies:
                c.wait()
            zf = jnp.zeros((16,), jnp.float32)
            for j in range(PRE_NMS_TOPN, _NSORT):
                rows[j, :] = zf
            pltpu.sync_copy(rows, out_hbm)


@functools.cache
def _make_sc_topk():
    @functools.partial(
        pl.kernel,
        out_type=jax.ShapeDtypeStruct((_NSORT, 16), jnp.float32),
        mesh=plsc.VectorSubcoreMesh(core_axis_name="c", subcore_axis_name="s"),
        compiler_params=pltpu.CompilerParams(needs_layout_passes=False,
                                             use_tc_tiling_on_sc=False),
        scratch_types=[
            pltpu.VMEM((_CHUNK,), jnp.int32),
            pltpu.VMEM((4096,), jnp.int32),
            pltpu.VMEM((16,), jnp.int32),
            pltpu.VMEM((_NTILES, 8), jnp.int32),
            pltpu.VMEM((_CHUNK,), jnp.int32),
            pltpu.VMEM((_CHUNK,), jnp.int32),
            pltpu.VMEM((_NCMP,), jnp.int32),
            pltpu.VMEM((_NCMP,), jnp.int32),
            pltpu.VMEM((_NCMP,), jnp.int32),
            pltpu.VMEM((_NCMP,), jnp.int32),
            pltpu.VMEM((4096,), jnp.int32),
            pltpu.VMEM((16, 128), jnp.int32),
            pltpu.VMEM((_NSORT, 16), jnp.float32),
            pltpu.VMEM_SHARED((_NTILES, 8), jnp.int32),
            pltpu.VMEM_SHARED((_NCMP,), jnp.int32),
            pltpu.VMEM_SHARED((_NCMP,), jnp.int32),
            pltpu.SemaphoreType.DMA,
        ],
    )
    def _sc_topk(k2_hbm, t_hbm, table_hbm, out_hbm, *scratch):
        _sc_topk_body(k2_hbm, t_hbm, table_hbm, out_hbm, *scratch)

    return _sc_topk


_P1 = 512


def _nms_block(rows4, cols_ref, b, ncols):
    x1r, y1r, x2r, y2r, area_r = rows4
    sl = pl.ds(b * _BLK, _BLK)
    x1c = cols_ref[sl, 0:1]
    y1c = cols_ref[sl, 1:2]
    x2c = cols_ref[sl, 2:3]
    y2c = cols_ref[sl, 3:4]
    area_c = (x2c - x1c + 1.0) * (y2c - y1c + 1.0)
    xx1 = jnp.maximum(x1c, x1r[:, :ncols])
    yy1 = jnp.maximum(y1c, y1r[:, :ncols])
    xx2 = jnp.minimum(x2c, x2r[:, :ncols])
    yy2 = jnp.minimum(y2c, y2r[:, :ncols])
    inter = jnp.maximum(xx2 - xx1 + 1.0, 0.0) * jnp.maximum(yy2 - yy1 + 1.0, 0.0)
    iou = inter / (area_c + area_r[:, :ncols] - inter)
    jg = b * _BLK + jax.lax.broadcasted_iota(jnp.int32, (_BLK, ncols), 0)
    ig = jax.lax.broadcasted_iota(jnp.int32, (_BLK, ncols), 1)
    sup = (iou > NMS_THRESH) & (jg < ig) & (ig < PRE_NMS_TOPN) & (jg < PRE_NMS_TOPN)
    return sup.astype(jnp.bfloat16), (jg <= ig).astype(jnp.bfloat16)


def _nms_body(rows_ref, cols_ref, out_ref, s_mat, lt_mat, pos_ref, kc_ref):
    x1r = rows_ref[0:1, :]
    y1r = rows_ref[1:2, :]
    x2r = rows_ref[2:3, :]
    y2r = rows_ref[3:4, :]
    area_r = (x2r - x1r + 1.0) * (y2r - y1r + 1.0)
    rows4 = (x1r, y1r, x2r, y2r, area_r)

    for b in range(_P1 // _BLK):
        s_blk, lt_blk = _nms_block(rows4, cols_ref, b, _P1)
        s_mat[pl.ds(b * _BLK, _BLK), pl.ds(0, _P1)] = s_blk
        lt_mat[pl.ds(b * _BLK, _BLK), pl.ds(0, _P1)] = lt_blk

    icol1 = jax.lax.broadcasted_iota(jnp.int32, (8, _P1), 1)
    keep0a = jnp.ones((8, _P1), jnp.float32)

    def cond_a(carry):
        _, changed, it = carry
        return changed & (it < _P1)

    def body_a(carry):
        keep, _, it = carry
        nk = keep
        for _ in range(4):
            sup = jnp.dot(nk.astype(jnp.bfloat16),
                          s_mat[pl.ds(0, _P1), pl.ds(0, _P1)],
                          preferred_element_type=jnp.float32)
            nk = jnp.where(sup < 0.5, 1.0, 0.0)
        changed = jnp.sum(jnp.abs(nk - keep)) > 0.0
        return nk, changed, it + 4

    keep_a, _, _ = jax.lax.while_loop(cond_a, body_a, (keep0a, True, 0))
    kept_a = jnp.sum(keep_a[0:1, :])

    @pl.when(kept_a >= float(POST_NMS_TOPN))
    def _fast():
        cumk = jnp.dot(keep_a.astype(jnp.bfloat16),
                       lt_mat[pl.ds(0, _P1), pl.ds(0, _P1)],
                       preferred_element_type=jnp.float32)
        pos_ref[:, 0:_P1] = jnp.where(keep_a > 0.5, cumk - 1.0, 1e9)
        kc_ref[0] = kept_a
        kc_ref[1] = 1.0

    @pl.when(kept_a < float(POST_NMS_TOPN))
    def _slow():
        for b in range(_N // _BLK):
            s_blk, lt_blk = _nms_block(rows4, cols_ref, b, _N)
            s_mat[pl.ds(b * _BLK, _BLK), :] = s_blk
            lt_mat[pl.ds(b * _BLK, _BLK), :] = lt_blk

        icol = jax.lax.broadcasted_iota(jnp.int32, (8, _N), 1)
        inb = (icol < PRE_NMS_TOPN).astype(jnp.float32)

        def cond(carry):
            _, changed, it = carry
            return changed & (it < _N)

        def body(carry):
            keep, _, it = carry
            sup = jnp.dot(keep.astype(jnp.bfloat16), s_mat[...],
                          preferred_element_type=jnp.float32)
            nk = jnp.where((sup < 0.5) & (icol < PRE_NMS_TOPN), 1.0, 0.0)
            changed = jnp.sum(jnp.abs(nk - keep)) > 0.0
            return nk, changed, it + 1

        keep, _, _ = jax.lax.while_loop(cond, body, (inb, True, 0))

        kcount = jnp.sum(keep[0:1, :])
        fill = (1.0 - keep) * inb
        cumk = jnp.dot(keep.astype(jnp.bfloat16), lt_mat[...],
                       preferred_element_type=jnp.float32)
        cumf = jnp.dot(fill.astype(jnp.bfloat16), lt_mat[...],
                       preferred_element_type=jnp.float32)
        pos = jnp.where(keep > 0.5, cumk - 1.0, kcount + cumf - 1.0)
        pos_ref[...] = jnp.where(inb > 0.5, pos, 1e9)
        kc_ref[0] = kcount
        kc_ref[1] = 0.0

    kcount = kc_ref[0]
    rvec = jax.lax.broadcasted_iota(jnp.int32, (_OUT_ROWS, 1), 0).astype(jnp.float32)

    def emit(ncols):
        pos1 = pos_ref[0:1, 0:ncols]
        riota = jax.lax.broadcasted_iota(
            jnp.int32, (_OUT_ROWS, ncols), 0).astype(jnp.float32)
        sel = riota == jnp.broadcast_to(pos1, (_OUT_ROWS, ncols))

        def pick(vals_row):
            v = jnp.broadcast_to(vals_row[:, 0:ncols], (_OUT_ROWS, ncols))
            return jnp.max(jnp.where(sel, v, -3.0e38), axis=1, keepdims=True)

        out_ref[:, 0:1] = jnp.zeros((_OUT_ROWS, 1), jnp.float32)
        out_ref[:, 1:2] = pick(x1r)
        out_ref[:, 2:3] = pick(y1r)
        out_ref[:, 3:4] = pick(x2r)
        out_ref[:, 4:5] = pick(y2r)
        scpick = pick(rows_ref[4:5, :])
        out_ref[:, 5:6] = jnp.where(rvec < kcount, scpick, _NEG)
        out_ref[:, 6:8] = jnp.zeros((_OUT_ROWS, 2), jnp.float32)

    fastf = kc_ref[1]

    @pl.when(fastf > 0.5)
    def _emit_fast():
        emit(_P1)

    @pl.when(fastf < 0.5)
    def _emit_slow():
        emit(_N)


def _nms(rows, cols):
    return pl.pallas_call(
        _nms_body,
        out_shape=jax.ShapeDtypeStruct((_OUT_ROWS, 8), jnp.float32),
        scratch_shapes=[
            pltpu.VMEM((_N, _N), jnp.bfloat16),
            pltpu.VMEM((_N, _N), jnp.bfloat16),
            pltpu.VMEM((8, _N), jnp.float32),
            pltpu.SMEM((2,), jnp.float32),
        ],
    )(rows, cols)


def kernel(scores, bbox_deltas, im_info, anchors):
    A = anchors.shape[0]
    H, W = scores.shape[2], scores.shape[3]
    fg = scores[0, A:].reshape(A, H * W)
    dl = bbox_deltas[0].reshape(A, 4, H * W)

    x1, y1, x2, y2, scm, key, tsel = _transform(fg, dl, anchors, im_info)

    def flat(v):
        return v.T.reshape(-1)

    npad = _NALL - A * H * W
    k2 = jnp.concatenate([flat(key), jnp.full((npad,), -1, jnp.int32)])
    tab = jnp.stack([x1, y1, x2, y2, scm], axis=-1)
    tab = jnp.pad(tab, ((0, 0), (0, 0), (0, 11))).reshape(A * H * W, 16)
    tab = jnp.pad(tab, ((0, npad), (0, 0)))

    sorted_tab = _make_sc_topk()(k2, tsel, tab)
    cols = sorted_tab[:, :8]
    rows = cols.T

    out = _nms(rows, cols)
    return out[:POST_NMS_TOPN, :6]


# speedup vs baseline: 1.9643x; 1.3411x over previous
import functools

import jax
import jax.numpy as jnp
from jax import lax
from jax.experimental import pallas as pl
from jax.experimental.pallas import tpu as pltpu
from jax.experimental.pallas import tpu_sc as plsc

FEAT_STRIDE = 16.0
PRE_NMS_TOPN = 2000
POST_NMS_TOPN = 300
NMS_THRESH = 0.7
MIN_SIZE = 16.0

_N = 2048
_BLK = 128
_OUT_ROWS = 384
_NEG = -1e9


def _transform_body(fg_ref, dl_ref, anch_ref, im_ref, x1_ref, y1_ref, x2_ref, y2_ref, sc_ref, key_ref, t_ref):
    hw = jax.lax.broadcasted_iota(jnp.int32, (9, 2500), 1).astype(jnp.float32)
    row = jnp.floor((hw + 0.5) * (1.0 / 50.0))
    sy = row * FEAT_STRIDE
    sx = (hw - 50.0 * row) * FEAT_STRIDE

    ax1 = anch_ref[:, 0:1] + sx
    ay1 = anch_ref[:, 1:2] + sy
    ax2 = anch_ref[:, 2:3] + sx
    ay2 = anch_ref[:, 3:4] + sy

    widths = ax2 - ax1 + 1.0
    heights = ay2 - ay1 + 1.0
    ctr_x = ax1 + 0.5 * widths
    ctr_y = ay1 + 0.5 * heights

    dx = dl_ref[:, 0, :]
    dy = dl_ref[:, 1, :]
    dw = dl_ref[:, 2, :]
    dh = dl_ref[:, 3, :]

    pred_ctr_x = dx * widths + ctr_x
    pred_ctr_y = dy * heights + ctr_y
    pred_w = jnp.exp(dw) * widths
    pred_h = jnp.exp(dh) * heights

    im_h = im_ref[0:1, 0:1]
    im_w = im_ref[0:1, 1:2]
    im_scale = im_ref[0:1, 2:3]

    x1 = jnp.clip(pred_ctr_x - 0.5 * pred_w, 0.0, im_w - 1.0)
    y1 = jnp.clip(pred_ctr_y - 0.5 * pred_h, 0.0, im_h - 1.0)
    x2 = jnp.clip(pred_ctr_x + 0.5 * pred_w, 0.0, im_w - 1.0)
    y2 = jnp.clip(pred_ctr_y + 0.5 * pred_h, 0.0, im_h - 1.0)

    ws = x2 - x1 + 1.0
    hs = y2 - y1 + 1.0
    min_size = MIN_SIZE * im_scale
    valid = (ws >= min_size) & (hs >= min_size)

    x1_ref[...] = x1
    y1_ref[...] = y1
    x2_ref[...] = x2
    y2_ref[...] = y2
    scm = jnp.where(valid, fg_ref[...], _NEG)
    sc_ref[...] = scm
    b = jax.lax.bitcast_convert_type(scm, jnp.int32)
    u = jnp.where(b >= 0, b ^ jnp.int32(-2147483648), ~b)
    k2 = ~u
    key_ref[...] = k2

    kx = k2 ^ jnp.int32(-2147483648)

    def bit_round(i, p):
        cand = p | jax.lax.shift_left(jnp.int32(1), 31 - i)
        candx = cand ^ jnp.int32(-2147483648)
        cnt = jnp.sum((kx < candx).astype(jnp.int32))
        return jnp.where(cnt < PRE_NMS_TOPN, cand, p)

    t_val = jax.lax.fori_loop(0, 32, bit_round, jnp.int32(0))
    t_ref[...] = jnp.zeros((8, 128), jnp.int32) + t_val


def _transform(fg, dl, anchors, im_info):
    out = jax.ShapeDtypeStruct((9, 2500), jnp.float32)
    outi = jax.ShapeDtypeStruct((9, 2500), jnp.int32)
    outt = jax.ShapeDtypeStruct((8, 128), jnp.int32)
    return pl.pallas_call(
        _transform_body,
        out_shape=(out, out, out, out, out, outi, outt),
    )(fg, dl, anchors, im_info)


_NALL = 22528
_NTILES = 16
_CHUNK = _NALL // _NTILES
_CVECS = _CHUNK // 16
_NCMP = 2304
_SLICE = _NCMP // _NTILES
_PAD_ROW = 2503


def _sc_topk_body(k2_hbm, t_hbm, table_hbm, out_hbm, keys_hbm,
                  lock2, cnt16, pcl, lbufk, lbufn, rows_t, pubcnt, skc,
                  snc, sem):
    cid = lax.axis_index("c")
    sid = lax.axis_index("s")
    lane = lax.broadcasted_iota(jnp.int32, (16,), 0)
    zeros16 = jnp.zeros((16,), jnp.int32)

    @pl.when(cid == 0)
    def _core0():
        base = sid * _CHUNK
        pltpu.sync_copy(k2_hbm.at[pl.ds(base, _CHUNK)], lock2)
        pltpu.sync_copy(t_hbm.at[0, pl.ds(0, 16)], cnt16)
        tv = jnp.max(cnt16[...])
        tx = tv ^ jnp.int32(-2147483648)

        def cnt_body(v, carry):
            ltc, eqc = carry
            kv = lock2[pl.ds(v * 16, 16)]
            lt = (kv ^ jnp.int32(-2147483648)) < tx
            eq = kv == tv
            return ltc + jnp.sum(lt.astype(jnp.int32)), eqc + jnp.sum(eq.astype(jnp.int32))
        lt_cnt, eq_cnt = lax.fori_loop(0, _CVECS, cnt_body,
                                       (jnp.int32(0), jnp.int32(0)))
        cnt16[...] = jnp.where(lane == 0, lt_cnt, jnp.where(lane == 1, eq_cnt, 0))
        pltpu.sync_copy(cnt16.at[pl.ds(0, 8)], pubcnt.at[sid])
        plsc.subcore_barrier()
        pltpu.sync_copy(pubcnt, pcl)
        lt_all = plsc.load_gather(pcl, [lane, zeros16])
        eq_all = plsc.load_gather(pcl, [lane, jnp.full((16,), 1, jnp.int32)])
        total_lt = jnp.sum(lt_all)
        take_total = jnp.int32(PRE_NMS_TOPN) - total_lt
        eq_excl = plsc.cumsum(eq_all) - eq_all
        take_all = jnp.clip(take_total - eq_excl, 0, eq_all)
        sel_all = lt_all + take_all
        padded_all = ((sel_all + 15) >> 4) << 4
        my_off = jnp.sum(jnp.where(lane < sid, padded_all, 0))
        my_take = jnp.sum(jnp.where(lane == sid, take_all, 0))
        my_pad = jnp.sum(jnp.where(lane == sid, padded_all, 0))
        total_padded = jnp.sum(padded_all)

        def comp_body(v, carry):
            cnt, eqc = carry
            kv = lock2[pl.ds(v * 16, 16)]
            lt = (kv ^ jnp.int32(-2147483648)) < tx
            eq = kv == tv
            eqi = eq.astype(jnp.int32)
            eqrank = eqc + plsc.cumsum(eqi) - 1
            take = lt | (eq & (eqrank < my_take))
            ti = take.astype(jnp.int32)
            posv = cnt + plsc.cumsum(ti) - 1
            plsc.store_scatter(lbufk, [posv], kv, mask=take)
            nvec = base + v * 16 + lane
            q = nvec // 9
            plsc.store_scatter(lbufn, [posv], (nvec - q * 9) * 2500 + q,
                               mask=take)
            return cnt + jnp.sum(ti), eqc + jnp.sum(eqi)
        my_sel, _ = lax.fori_loop(0, _CVECS, comp_body,
                                  (jnp.int32(0), jnp.int32(0)))
        padmask = lane < (my_pad - my_sel)
        plsc.store_scatter(lbufk, [my_sel + lane],
                           jnp.full((16,), -1, jnp.int32), mask=padmask)
        plsc.store_scatter(lbufn, [my_sel + lane],
                           jnp.full((16,), _PAD_ROW, jnp.int32), mask=padmask)

        def dma_body(c, _):
            off = pl.multiple_of(my_off + c * 16, 16)

            @pl.when(c * 16 < my_pad)
            def _():
                pltpu.sync_copy(lbufk.at[pl.ds(c * 16, 16)],
                                skc.at[pl.ds(off, 16)])
                pltpu.sync_copy(lbufn.at[pl.ds(c * 16, 16)],
                                snc.at[pl.ds(off, 16)])
            return 0
        lax.fori_loop(0, _CVECS, dma_body, 0)

        @pl.when(sid == 0)
        def _tail_fill():
            lbufk[pl.ds(0, 16)] = jnp.full((16,), -1, jnp.int32)
            lbufn[pl.ds(0, 16)] = jnp.full((16,), _PAD_ROW, jnp.int32)

            def tf(c, _):
                pos = total_padded + c * 16

                @pl.when(pos < _NCMP)
                def _():
                    off = pl.multiple_of(pos, 16)
                    pltpu.sync_copy(lbufk.at[pl.ds(0, 16)],
                                    skc.at[pl.ds(off, 16)])
                    pltpu.sync_copy(lbufn.at[pl.ds(0, 16)],
                                    snc.at[pl.ds(off, 16)])
                return 0
            lax.fori_loop(0, (_NCMP - PRE_NMS_TOPN) // 16 + 1, tf, 0)

        plsc.subcore_barrier()

        pltpu.sync_copy(snc.at[pl.ds(sid * _SLICE, _SLICE)],
                        lbufn.at[pl.ds(0, _SLICE)])
        g1 = pltpu.async_copy(table_hbm.at[lbufn.at[pl.ds(0, 128)]],
                              rows_t.at[pl.ds(0, 128)], sem)
        g2 = pltpu.async_copy(table_hbm.at[lbufn.at[pl.ds(128, 16)]],
                              rows_t.at[pl.ds(128, 16)], sem)
        g1.wait()
        g2.wait()
        pltpu.sync_copy(rows_t, out_hbm.at[pl.ds(sid * _SLICE, _SLICE)])

        @pl.when(sid == 0)
        def _keys_out():
            pltpu.sync_copy(skc, keys_hbm)


@functools.cache
def _make_sc_topk():
    @functools.partial(
        pl.kernel,
        out_type=(jax.ShapeDtypeStruct((_NCMP, 16), jnp.float32),
                  jax.ShapeDtypeStruct((_NCMP,), jnp.int32)),
        mesh=plsc.VectorSubcoreMesh(core_axis_name="c", subcore_axis_name="s"),
        compiler_params=pltpu.CompilerParams(needs_layout_passes=False,
                                             use_tc_tiling_on_sc=False),
        scratch_types=[
            pltpu.VMEM((_CHUNK,), jnp.int32),
            pltpu.VMEM((16,), jnp.int32),
            pltpu.VMEM((_NTILES, 8), jnp.int32),
            pltpu.VMEM((_CHUNK,), jnp.int32),
            pltpu.VMEM((_CHUNK,), jnp.int32),
            pltpu.VMEM((_SLICE, 16), jnp.float32),
            pltpu.VMEM_SHARED((_NTILES, 8), jnp.int32),
            pltpu.VMEM_SHARED((_NCMP,), jnp.int32),
            pltpu.VMEM_SHARED((_NCMP,), jnp.int32),
            pltpu.SemaphoreType.DMA,
        ],
    )
    def _sc_topk(k2_hbm, t_hbm, table_hbm, out_hbm, keys_hbm, *scratch):
        _sc_topk_body(k2_hbm, t_hbm, table_hbm, out_hbm, keys_hbm, *scratch)

    return _sc_topk


_P1 = 512
_NC = _NCMP


def _nms_block(rows4, cols_ref, b, ncols):
    x1r, y1r, x2r, y2r, area_r = rows4
    sl = pl.ds(b * _BLK, _BLK)
    x1c = cols_ref[sl, 0:1]
    y1c = cols_ref[sl, 1:2]
    x2c = cols_ref[sl, 2:3]
    y2c = cols_ref[sl, 3:4]
    area_c = (x2c - x1c + 1.0) * (y2c - y1c + 1.0)
    xx1 = jnp.maximum(x1c, x1r[:, :ncols])
    yy1 = jnp.maximum(y1c, y1r[:, :ncols])
    xx2 = jnp.minimum(x2c, x2r[:, :ncols])
    yy2 = jnp.minimum(y2c, y2r[:, :ncols])
    inter = jnp.maximum(xx2 - xx1 + 1.0, 0.0) * jnp.maximum(yy2 - yy1 + 1.0, 0.0)
    iou = inter / (area_c + area_r[:, :ncols] - inter)
    jg = b * _BLK + jax.lax.broadcasted_iota(jnp.int32, (_BLK, ncols), 0)
    ig = jax.lax.broadcasted_iota(jnp.int32, (_BLK, ncols), 1)
    sup = (iou > NMS_THRESH) & (jg < ig) & (ig < PRE_NMS_TOPN) & (jg < PRE_NMS_TOPN)
    return sup.astype(jnp.bfloat16), (jg <= ig).astype(jnp.bfloat16)


def _nms_body(tabc_ref, tabr_ref, keyc_ref, keyr_ref, out_ref,
              s_mat, lt_mat, pos_ref, kc_ref, colsb, rowsb, p_mat, rk_ref):
    mini = jnp.int32(-2147483648)
    kxr = keyr_ref[...] ^ mini
    for b in range(_NC // _BLK):
        kxc = keyc_ref[pl.ds(b * _BLK, _BLK), :] ^ mini
        jg = b * _BLK + jax.lax.broadcasted_iota(jnp.int32, (_BLK, _NC), 0)
        ig = jax.lax.broadcasted_iota(jnp.int32, (_BLK, _NC), 1)
        lt = kxc < kxr
        eq = kxc == kxr
        cmp = (lt | (eq & (jg < ig))).astype(jnp.float32)
        part = jnp.sum(cmp, axis=0, keepdims=True)
        if b == 0:
            rk_ref[...] = part
        else:
            rk_ref[...] = rk_ref[...] + part

    def build_sorted(nr):
        for b in range(nr // _BLK):
            ri = (b * _BLK + jax.lax.broadcasted_iota(
                jnp.int32, (_BLK, _NC), 0)).astype(jnp.float32)
            p_mat[pl.ds(b * _BLK, _BLK), :] = (
                ri == jnp.broadcast_to(rk_ref[...], (_BLK, _NC))
            ).astype(jnp.bfloat16)
        tc = tabc_ref[...]
        h1 = tc.astype(jnp.bfloat16)
        r1 = tc - h1.astype(jnp.float32)
        h2 = r1.astype(jnp.bfloat16)
        r2 = r1 - h2.astype(jnp.float32)
        h3 = r2.astype(jnp.bfloat16)
        for b in range(max(nr // 512, 1)):
            bw = min(nr, 512)
            pm = p_mat[pl.ds(b * 512, 512), :] if bw == 512 else p_mat[0:nr, :]
            cs = (jnp.dot(pm, h1, preferred_element_type=jnp.float32)
                  + jnp.dot(pm, h2, preferred_element_type=jnp.float32)
                  + jnp.dot(pm, h3, preferred_element_type=jnp.float32))
            colsb[pl.ds(b * 512, bw), :] = cs[0:bw, :]
            rowsb[:, pl.ds(b * 512, bw)] = jnp.transpose(cs[0:bw, :], (1, 0))

    def rows4_now():
        x1r = rowsb[0:1, :]
        y1r = rowsb[1:2, :]
        x2r = rowsb[2:3, :]
        y2r = rowsb[3:4, :]
        area_r = (x2r - x1r + 1.0) * (y2r - y1r + 1.0)
        return (x1r, y1r, x2r, y2r, area_r)

    build_sorted(_P1)
    rows4 = rows4_now()
    for b in range(_P1 // _BLK):
        s_blk, lt_blk = _nms_block(rows4, colsb, b, _P1)
        s_mat[pl.ds(b * _BLK, _BLK), pl.ds(0, _P1)] = s_blk
        lt_mat[pl.ds(b * _BLK, _BLK), pl.ds(0, _P1)] = lt_blk

    icol1 = jax.lax.broadcasted_iota(jnp.int32, (8, _P1), 1)
    keep0a = jnp.ones((8, _P1), jnp.float32)

    def cond_a(carry):
        _, changed, it = carry
        return changed & (it < _P1)

    def body_a(carry):
        keep, _, it = carry
        nk = keep
        for _ in range(4):
            sup = jnp.dot(nk.astype(jnp.bfloat16),
                          s_mat[pl.ds(0, _P1), pl.ds(0, _P1)],
                          preferred_element_type=jnp.float32)
            nk = jnp.where(sup < 0.5, 1.0, 0.0)
        changed = jnp.sum(jnp.abs(nk - keep)) > 0.0
        return nk, changed, it + 4

    keep_a, _, _ = jax.lax.while_loop(cond_a, body_a, (keep0a, True, 0))
    kept_a = jnp.sum(keep_a[0:1, :])

    @pl.when(kept_a >= float(POST_NMS_TOPN))
    def _fast():
        cumk = jnp.dot(keep_a.astype(jnp.bfloat16),
                       lt_mat[pl.ds(0, _P1), pl.ds(0, _P1)],
                       preferred_element_type=jnp.float32)
        pos_ref[:, 0:_P1] = jnp.where(keep_a > 0.5, cumk - 1.0, 1e9)
        kc_ref[0] = kept_a
        kc_ref[1] = 1.0

    @pl.when(kept_a < float(POST_NMS_TOPN))
    def _slow():
        build_sorted(_N)
        rows4b = rows4_now()
        for b in range(_N // _BLK):
            s_blk, lt_blk = _nms_block(rows4b, colsb, b, _N)
            s_mat[pl.ds(b * _BLK, _BLK), :] = s_blk
            lt_mat[pl.ds(b * _BLK, _BLK), :] = lt_blk

        icol = jax.lax.broadcasted_iota(jnp.int32, (8, _N), 1)
        inb = (icol < PRE_NMS_TOPN).astype(jnp.float32)

        def cond(carry):
            _, changed, it = carry
            return changed & (it < _N)

        def body(carry):
            keep, _, it = carry
            sup = jnp.dot(keep.astype(jnp.bfloat16), s_mat[...],
                          preferred_element_type=jnp.float32)
            nk = jnp.where((sup < 0.5) & (icol < PRE_NMS_TOPN), 1.0, 0.0)
            changed = jnp.sum(jnp.abs(nk - keep)) > 0.0
            return nk, changed, it + 1

        keep, _, _ = jax.lax.while_loop(cond, body, (inb, True, 0))

        kcount = jnp.sum(keep[0:1, :])
        fill = (1.0 - keep) * inb
        cumk = jnp.dot(keep.astype(jnp.bfloat16), lt_mat[...],
                       preferred_element_type=jnp.float32)
        cumf = jnp.dot(fill.astype(jnp.bfloat16), lt_mat[...],
                       preferred_element_type=jnp.float32)
        pos = jnp.where(keep > 0.5, cumk - 1.0, kcount + cumf - 1.0)
        pos_ref[...] = jnp.where(inb > 0.5, pos, 1e9)
        kc_ref[0] = kcount
        kc_ref[1] = 0.0

    kcount = kc_ref[0]
    rvec = jax.lax.broadcasted_iota(jnp.int32, (_OUT_ROWS, 1), 0).astype(jnp.float32)

    def emit(ncols):
        pos1 = pos_ref[0:1, 0:ncols]
        riota = jax.lax.broadcasted_iota(
            jnp.int32, (_OUT_ROWS, ncols), 0).astype(jnp.float32)
        sel = riota == jnp.broadcast_to(pos1, (_OUT_ROWS, ncols))

        def pick(vals_row):
            v = jnp.broadcast_to(vals_row[:, 0:ncols], (_OUT_ROWS, ncols))
            return jnp.max(jnp.where(sel, v, -3.0e38), axis=1, keepdims=True)

        out_ref[:, 0:1] = jnp.zeros((_OUT_ROWS, 1), jnp.float32)
        out_ref[:, 1:2] = pick(rowsb[0:1, :])
        out_ref[:, 2:3] = pick(rowsb[1:2, :])
        out_ref[:, 3:4] = pick(rowsb[2:3, :])
        out_ref[:, 4:5] = pick(rowsb[3:4, :])
        scpick = pick(rowsb[4:5, :])
        out_ref[:, 5:6] = jnp.where(rvec < kcount, scpick, _NEG)
        out_ref[:, 6:8] = jnp.zeros((_OUT_ROWS, 2), jnp.float32)

    fastf = kc_ref[1]

    @pl.when(fastf > 0.5)
    def _emit_fast():
        emit(_P1)

    @pl.when(fastf < 0.5)
    def _emit_slow():
        emit(_N)


def _nms(tabc, tabr, keyc, keyr):
    return pl.pallas_call(
        _nms_body,
        out_shape=jax.ShapeDtypeStruct((_OUT_ROWS, 8), jnp.float32),
        scratch_shapes=[
            pltpu.VMEM((_N, _N), jnp.bfloat16),
            pltpu.VMEM((_N, _N), jnp.bfloat16),
            pltpu.VMEM((8, _N), jnp.float32),
            pltpu.SMEM((2,), jnp.float32),
            pltpu.VMEM((_N, 8), jnp.float32),
            pltpu.VMEM((8, _N), jnp.float32),
            pltpu.VMEM((_N, _NC), jnp.bfloat16),
            pltpu.VMEM((1, _NC), jnp.float32),
        ],
    )(tabc, tabr, keyc, keyr)


def kernel(scores, bbox_deltas, im_info, anchors):
    A = anchors.shape[0]
    H, W = scores.shape[2], scores.shape[3]
    fg = scores[0, A:].reshape(A, H * W)
    dl = bbox_deltas[0].reshape(A, 4, H * W)

    x1, y1, x2, y2, scm, key, tsel = _transform(fg, dl, anchors, im_info)

    def flat(v):
        return v.T.reshape(-1)

    npad = _NALL - A * H * W
    k2 = jnp.concatenate([flat(key), jnp.full((npad,), -1, jnp.int32)])
    tab = jnp.stack([x1, y1, x2, y2, scm], axis=-1)
    tab = jnp.pad(tab, ((0, 0), (0, 0), (0, 11))).reshape(A * H * W, 16)
    tab = jnp.pad(tab, ((0, npad), (0, 0)))

    comp_tab, keys = _make_sc_topk()(k2, tsel, tab)
    tabc = comp_tab[:, :8]

    out = _nms(tabc, tabc.T, keys.reshape(_NCMP, 1), keys.reshape(1, _NCMP))
    return out[:POST_NMS_TOPN, :6]


# speedup vs baseline: 1.9644x; 1.0000x over previous
import functools

import jax
import jax.numpy as jnp
from jax import lax
from jax.experimental import pallas as pl
from jax.experimental.pallas import tpu as pltpu
from jax.experimental.pallas import tpu_sc as plsc

FEAT_STRIDE = 16.0
PRE_NMS_TOPN = 2000
POST_NMS_TOPN = 300
NMS_THRESH = 0.7
MIN_SIZE = 16.0

_N = 2048
_BLK = 128
_OUT_ROWS = 384
_NEG = -1e9


def _transform_body(fg_ref, dl_ref, anch_ref, im_ref, x1_ref, y1_ref, x2_ref, y2_ref, sc_ref, key_ref, t_ref):
    hw = jax.lax.broadcasted_iota(jnp.int32, (9, 2500), 1).astype(jnp.float32)
    row = jnp.floor((hw + 0.5) * (1.0 / 50.0))
    sy = row * FEAT_STRIDE
    sx = (hw - 50.0 * row) * FEAT_STRIDE

    ax1 = anch_ref[:, 0:1] + sx
    ay1 = anch_ref[:, 1:2] + sy
    ax2 = anch_ref[:, 2:3] + sx
    ay2 = anch_ref[:, 3:4] + sy

    widths = ax2 - ax1 + 1.0
    heights = ay2 - ay1 + 1.0
    ctr_x = ax1 + 0.5 * widths
    ctr_y = ay1 + 0.5 * heights

    dx = dl_ref[:, 0, :]
    dy = dl_ref[:, 1, :]
    dw = dl_ref[:, 2, :]
    dh = dl_ref[:, 3, :]

    pred_ctr_x = dx * widths + ctr_x
    pred_ctr_y = dy * heights + ctr_y
    pred_w = jnp.exp(dw) * widths
    pred_h = jnp.exp(dh) * heights

    im_h = im_ref[0:1, 0:1]
    im_w = im_ref[0:1, 1:2]
    im_scale = im_ref[0:1, 2:3]

    x1 = jnp.clip(pred_ctr_x - 0.5 * pred_w, 0.0, im_w - 1.0)
    y1 = jnp.clip(pred_ctr_y - 0.5 * pred_h, 0.0, im_h - 1.0)
    x2 = jnp.clip(pred_ctr_x + 0.5 * pred_w, 0.0, im_w - 1.0)
    y2 = jnp.clip(pred_ctr_y + 0.5 * pred_h, 0.0, im_h - 1.0)

    ws = x2 - x1 + 1.0
    hs = y2 - y1 + 1.0
    min_size = MIN_SIZE * im_scale
    valid = (ws >= min_size) & (hs >= min_size)

    x1_ref[...] = x1
    y1_ref[...] = y1
    x2_ref[...] = x2
    y2_ref[...] = y2
    scm = jnp.where(valid, fg_ref[...], _NEG)
    sc_ref[...] = scm
    b = jax.lax.bitcast_convert_type(scm, jnp.int32)
    u = jnp.where(b >= 0, b ^ jnp.int32(-2147483648), ~b)
    k2 = ~u
    key_ref[...] = k2

    kx = k2 ^ jnp.int32(-2147483648)

    def bit_round(i, p):
        cand = p | jax.lax.shift_left(jnp.int32(1), 31 - i)
        candx = cand ^ jnp.int32(-2147483648)
        cnt = jnp.sum((kx < candx).astype(jnp.int32))
        return jnp.where(cnt < PRE_NMS_TOPN, cand, p)

    t_val = jax.lax.fori_loop(0, 32, bit_round, jnp.int32(0))
    t_ref[...] = jnp.zeros((8, 128), jnp.int32) + t_val


def _transform(fg, dl, anchors, im_info):
    out = jax.ShapeDtypeStruct((9, 2500), jnp.float32)
    outi = jax.ShapeDtypeStruct((9, 2500), jnp.int32)
    outt = jax.ShapeDtypeStruct((8, 128), jnp.int32)
    return pl.pallas_call(
        _transform_body,
        out_shape=(out, out, out, out, out, outi, outt),
    )(fg, dl, anchors, im_info)


_NALL = 22528
_NTILES = 16
_CHUNK = _NALL // _NTILES
_CVECS = _CHUNK // 16
_NCMP = 2304
_SLICE = _NCMP // _NTILES
_PAD_ROW = 2503


def _sc_topk_body(k2_hbm, t_hbm, table_hbm, out_hbm, keys_hbm,
                  lock2, cnt16, pcl, lbufk, lbufn, rows_t, pubcnt, skc,
                  snc, sem):
    cid = lax.axis_index("c")
    sid = lax.axis_index("s")
    lane = lax.broadcasted_iota(jnp.int32, (16,), 0)
    zeros16 = jnp.zeros((16,), jnp.int32)

    @pl.when(cid == 0)
    def _core0():
        base = sid * _CHUNK
        pltpu.sync_copy(k2_hbm.at[pl.ds(base, _CHUNK)], lock2)
        pltpu.sync_copy(t_hbm.at[0, pl.ds(0, 16)], cnt16)
        tv = jnp.max(cnt16[...])
        tx = tv ^ jnp.int32(-2147483648)

        def cnt_body(v, carry):
            ltc, eqc = carry
            kv = lock2[pl.ds(v * 16, 16)]
            lt = (kv ^ jnp.int32(-2147483648)) < tx
            eq = kv == tv
            return ltc + jnp.sum(lt.astype(jnp.int32)), eqc + jnp.sum(eq.astype(jnp.int32))
        lt_cnt, eq_cnt = lax.fori_loop(0, _CVECS, cnt_body,
                                       (jnp.int32(0), jnp.int32(0)))
        cnt16[...] = jnp.where(lane == 0, lt_cnt, jnp.where(lane == 1, eq_cnt, 0))
        pltpu.sync_copy(cnt16.at[pl.ds(0, 8)], pubcnt.at[sid])
        plsc.subcore_barrier()
        pltpu.sync_copy(pubcnt, pcl)
        lt_all = plsc.load_gather(pcl, [lane, zeros16])
        eq_all = plsc.load_gather(pcl, [lane, jnp.full((16,), 1, jnp.int32)])
        total_lt = jnp.sum(lt_all)
        take_total = jnp.int32(PRE_NMS_TOPN) - total_lt
        eq_excl = plsc.cumsum(eq_all) - eq_all
        take_all = jnp.clip(take_total - eq_excl, 0, eq_all)
        sel_all = lt_all + take_all
        padded_all = ((sel_all + 15) >> 4) << 4
        my_off = jnp.sum(jnp.where(lane < sid, padded_all, 0))
        my_take = jnp.sum(jnp.where(lane == sid, take_all, 0))
        my_pad = jnp.sum(jnp.where(lane == sid, padded_all, 0))
        total_padded = jnp.sum(padded_all)

        def comp_body(v, carry):
            cnt, eqc = carry
            kv = lock2[pl.ds(v * 16, 16)]
            lt = (kv ^ jnp.int32(-2147483648)) < tx
            eq = kv == tv
            eqi = eq.astype(jnp.int32)
            eqrank = eqc + plsc.cumsum(eqi) - 1
            take = lt | (eq & (eqrank < my_take))
            ti = take.astype(jnp.int32)
            posv = cnt + plsc.cumsum(ti) - 1
            plsc.store_scatter(lbufk, [posv], kv, mask=take)
            nvec = base + v * 16 + lane
            q = nvec // 9
            plsc.store_scatter(lbufn, [posv], (nvec - q * 9) * 2500 + q,
                               mask=take)
            return cnt + jnp.sum(ti), eqc + jnp.sum(eqi)
        my_sel, _ = lax.fori_loop(0, _CVECS, comp_body,
                                  (jnp.int32(0), jnp.int32(0)))
        padmask = lane < (my_pad - my_sel)
        plsc.store_scatter(lbufk, [my_sel + lane],
                           jnp.full((16,), -1, jnp.int32), mask=padmask)
        plsc.store_scatter(lbufn, [my_sel + lane],
                           jnp.full((16,), _PAD_ROW, jnp.int32), mask=padmask)

        def dma_body(c, _):
            off = pl.multiple_of(my_off + c * 16, 16)

            @pl.when(c * 16 < my_pad)
            def _():
                pltpu.sync_copy(lbufk.at[pl.ds(c * 16, 16)],
                                skc.at[pl.ds(off, 16)])
                pltpu.sync_copy(lbufn.at[pl.ds(c * 16, 16)],
                                snc.at[pl.ds(off, 16)])
            return 0
        lax.fori_loop(0, _CVECS, dma_body, 0)

        @pl.when(sid == 0)
        def _tail_fill():
            lbufk[pl.ds(0, 16)] = jnp.full((16,), -1, jnp.int32)
            lbufn[pl.ds(0, 16)] = jnp.full((16,), _PAD_ROW, jnp.int32)

            def tf(c, _):
                pos = total_padded + c * 16

                @pl.when(pos < _NCMP)
                def _():
                    off = pl.multiple_of(pos, 16)
                    pltpu.sync_copy(lbufk.at[pl.ds(0, 16)],
                                    skc.at[pl.ds(off, 16)])
                    pltpu.sync_copy(lbufn.at[pl.ds(0, 16)],
                                    snc.at[pl.ds(off, 16)])
                return 0
            lax.fori_loop(0, (_NCMP - PRE_NMS_TOPN) // 16 + 1, tf, 0)

        plsc.subcore_barrier()

        pltpu.sync_copy(snc.at[pl.ds(sid * _SLICE, _SLICE)],
                        lbufn.at[pl.ds(0, _SLICE)])
        g1 = pltpu.async_copy(table_hbm.at[lbufn.at[pl.ds(0, 128)]],
                              rows_t.at[pl.ds(0, 128)], sem)
        g2 = pltpu.async_copy(table_hbm.at[lbufn.at[pl.ds(128, 16)]],
                              rows_t.at[pl.ds(128, 16)], sem)
        g1.wait()
        g2.wait()
        pltpu.sync_copy(rows_t, out_hbm.at[pl.ds(sid * _SLICE, _SLICE)])

        @pl.when(sid == 0)
        def _keys_out():
            pltpu.sync_copy(skc, keys_hbm)


@functools.cache
def _make_sc_topk():
    @functools.partial(
        pl.kernel,
        out_type=(jax.ShapeDtypeStruct((_NCMP, 16), jnp.float32),
                  jax.ShapeDtypeStruct((_NCMP,), jnp.int32)),
        mesh=plsc.VectorSubcoreMesh(core_axis_name="c", subcore_axis_name="s"),
        compiler_params=pltpu.CompilerParams(needs_layout_passes=False,
                                             use_tc_tiling_on_sc=False),
        scratch_types=[
            pltpu.VMEM((_CHUNK,), jnp.int32),
            pltpu.VMEM((16,), jnp.int32),
            pltpu.VMEM((_NTILES, 8), jnp.int32),
            pltpu.VMEM((_CHUNK,), jnp.int32),
            pltpu.VMEM((_CHUNK,), jnp.int32),
            pltpu.VMEM((_SLICE, 16), jnp.float32),
            pltpu.VMEM_SHARED((_NTILES, 8), jnp.int32),
            pltpu.VMEM_SHARED((_NCMP,), jnp.int32),
            pltpu.VMEM_SHARED((_NCMP,), jnp.int32),
            pltpu.SemaphoreType.DMA,
        ],
    )
    def _sc_topk(k2_hbm, t_hbm, table_hbm, out_hbm, keys_hbm, *scratch):
        _sc_topk_body(k2_hbm, t_hbm, table_hbm, out_hbm, keys_hbm, *scratch)

    return _sc_topk


_P1 = 512
_NC = _NCMP


def _nms_block(rows4, cols_ref, b, ncols):
    x1r, y1r, x2r, y2r, area_r = rows4
    sl = pl.ds(b * _BLK, _BLK)
    x1c = cols_ref[sl, 0:1]
    y1c = cols_ref[sl, 1:2]
    x2c = cols_ref[sl, 2:3]
    y2c = cols_ref[sl, 3:4]
    area_c = (x2c - x1c + 1.0) * (y2c - y1c + 1.0)
    xx1 = jnp.maximum(x1c, x1r[:, :ncols])
    yy1 = jnp.maximum(y1c, y1r[:, :ncols])
    xx2 = jnp.minimum(x2c, x2r[:, :ncols])
    yy2 = jnp.minimum(y2c, y2r[:, :ncols])
    inter = jnp.maximum(xx2 - xx1 + 1.0, 0.0) * jnp.maximum(yy2 - yy1 + 1.0, 0.0)
    iou = inter / (area_c + area_r[:, :ncols] - inter)
    jg = b * _BLK + jax.lax.broadcasted_iota(jnp.int32, (_BLK, ncols), 0)
    ig = jax.lax.broadcasted_iota(jnp.int32, (_BLK, ncols), 1)
    sup = (iou > NMS_THRESH) & (jg < ig) & (ig < PRE_NMS_TOPN) & (jg < PRE_NMS_TOPN)
    return sup.astype(jnp.bfloat16), (jg <= ig).astype(jnp.bfloat16)


def _nms_body(tabc_ref, tabr_ref, keyc_ref, keyr_ref, out_ref,
              s_mat, lt_mat, pos_ref, kc_ref, colsb, rowsb, p_mat, rk_ref):
    mini = jnp.int32(-2147483648)
    kxr = keyr_ref[...] ^ mini
    for b in range(_NC // _BLK):
        kxc = keyc_ref[pl.ds(b * _BLK, _BLK), :] ^ mini
        jg = b * _BLK + jax.lax.broadcasted_iota(jnp.int32, (_BLK, _NC), 0)
        ig = jax.lax.broadcasted_iota(jnp.int32, (_BLK, _NC), 1)
        lt = kxc < kxr
        eq = kxc == kxr
        cmp = (lt | (eq & (jg < ig))).astype(jnp.float32)
        part = jnp.sum(cmp, axis=0, keepdims=True)
        if b == 0:
            rk_ref[...] = part
        else:
            rk_ref[...] = rk_ref[...] + part

    def build_sorted(nr):
        for b in range(nr // _BLK):
            ri = (b * _BLK + jax.lax.broadcasted_iota(
                jnp.int32, (_BLK, _NC), 0)).astype(jnp.float32)
            p_mat[pl.ds(b * _BLK, _BLK), :] = (
                ri == jnp.broadcast_to(rk_ref[...], (_BLK, _NC))
            ).astype(jnp.bfloat16)
        tc = tabc_ref[...]
        h1 = tc.astype(jnp.bfloat16)
        r1 = tc - h1.astype(jnp.float32)
        h2 = r1.astype(jnp.bfloat16)
        r2 = r1 - h2.astype(jnp.float32)
        h3 = r2.astype(jnp.bfloat16)
        for b in range(max(nr // 512, 1)):
            bw = min(nr, 512)
            pm = p_mat[pl.ds(b * 512, 512), :] if bw == 512 else p_mat[0:nr, :]
            cs = ((jnp.dot(pm, h2, preferred_element_type=jnp.float32)
                   + jnp.dot(pm, h3, preferred_element_type=jnp.float32))
                  + jnp.dot(pm, h1, preferred_element_type=jnp.float32))
            colsb[pl.ds(b * 512, bw), :] = cs[0:bw, :]
            rowsb[:, pl.ds(b * 512, bw)] = jnp.transpose(cs[0:bw, :], (1, 0))

    def rows4_now():
        x1r = rowsb[0:1, :]
        y1r = rowsb[1:2, :]
        x2r = rowsb[2:3, :]
        y2r = rowsb[3:4, :]
        area_r = (x2r - x1r + 1.0) * (y2r - y1r + 1.0)
        return (x1r, y1r, x2r, y2r, area_r)

    build_sorted(_P1)
    rows4 = rows4_now()
    for b in range(_P1 // _BLK):
        s_blk, lt_blk = _nms_block(rows4, colsb, b, _P1)
        s_mat[pl.ds(b * _BLK, _BLK), pl.ds(0, _P1)] = s_blk
        lt_mat[pl.ds(b * _BLK, _BLK), pl.ds(0, _P1)] = lt_blk

    icol1 = jax.lax.broadcasted_iota(jnp.int32, (8, _P1), 1)
    keep0a = jnp.ones((8, _P1), jnp.float32)

    def cond_a(carry):
        _, changed, it = carry
        return changed & (it < _P1)

    def body_a(carry):
        keep, _, it = carry
        nk = keep
        for _ in range(4):
            sup = jnp.dot(nk.astype(jnp.bfloat16),
                          s_mat[pl.ds(0, _P1), pl.ds(0, _P1)],
                          preferred_element_type=jnp.float32)
            nk = jnp.where(sup < 0.5, 1.0, 0.0)
        changed = jnp.sum(jnp.abs(nk - keep)) > 0.0
        return nk, changed, it + 4

    keep_a, _, _ = jax.lax.while_loop(cond_a, body_a, (keep0a, True, 0))
    kept_a = jnp.sum(keep_a[0:1, :])

    @pl.when(kept_a >= float(POST_NMS_TOPN))
    def _fast():
        cumk = jnp.dot(keep_a.astype(jnp.bfloat16),
                       lt_mat[pl.ds(0, _P1), pl.ds(0, _P1)],
                       preferred_element_type=jnp.float32)
        pos_ref[:, 0:_P1] = jnp.where(keep_a > 0.5, cumk - 1.0, 1e9)
        kc_ref[0] = kept_a
        kc_ref[1] = 1.0

    @pl.when(kept_a < float(POST_NMS_TOPN))
    def _slow():
        build_sorted(_N)
        rows4b = rows4_now()
        for b in range(_N // _BLK):
            s_blk, lt_blk = _nms_block(rows4b, colsb, b, _N)
            s_mat[pl.ds(b * _BLK, _BLK), :] = s_blk
            lt_mat[pl.ds(b * _BLK, _BLK), :] = lt_blk

        icol = jax.lax.broadcasted_iota(jnp.int32, (8, _N), 1)
        inb = (icol < PRE_NMS_TOPN).astype(jnp.float32)

        def cond(carry):
            _, changed, it = carry
            return changed & (it < _N)

        def body(carry):
            keep, _, it = carry
            sup = jnp.dot(keep.astype(jnp.bfloat16), s_mat[...],
                          preferred_element_type=jnp.float32)
            nk = jnp.where((sup < 0.5) & (icol < PRE_NMS_TOPN), 1.0, 0.0)
            changed = jnp.sum(jnp.abs(nk - keep)) > 0.0
            return nk, changed, it + 1

        keep, _, _ = jax.lax.while_loop(cond, body, (inb, True, 0))

        kcount = jnp.sum(keep[0:1, :])
        fill = (1.0 - keep) * inb
        cumk = jnp.dot(keep.astype(jnp.bfloat16), lt_mat[...],
                       preferred_element_type=jnp.float32)
        cumf = jnp.dot(fill.astype(jnp.bfloat16), lt_mat[...],
                       preferred_element_type=jnp.float32)
        pos = jnp.where(keep > 0.5, cumk - 1.0, kcount + cumf - 1.0)
        pos_ref[...] = jnp.where(inb > 0.5, pos, 1e9)
        kc_ref[0] = kcount
        kc_ref[1] = 0.0

    kcount = kc_ref[0]
    rvec = jax.lax.broadcasted_iota(jnp.int32, (_OUT_ROWS, 1), 0).astype(jnp.float32)

    def emit(ncols):
        pos1 = pos_ref[0:1, 0:ncols]
        riota = jax.lax.broadcasted_iota(
            jnp.int32, (_OUT_ROWS, ncols), 0).astype(jnp.float32)
        sel = riota == jnp.broadcast_to(pos1, (_OUT_ROWS, ncols))

        def pick(vals_row):
            v = jnp.broadcast_to(vals_row[:, 0:ncols], (_OUT_ROWS, ncols))
            return jnp.max(jnp.where(sel, v, -3.0e38), axis=1, keepdims=True)

        out_ref[:, 0:1] = jnp.zeros((_OUT_ROWS, 1), jnp.float32)
        out_ref[:, 1:2] = pick(rowsb[0:1, :])
        out_ref[:, 2:3] = pick(rowsb[1:2, :])
        out_ref[:, 3:4] = pick(rowsb[2:3, :])
        out_ref[:, 4:5] = pick(rowsb[3:4, :])
        scpick = pick(rowsb[4:5, :])
        out_ref[:, 5:6] = jnp.where(rvec < kcount, scpick, _NEG)
        out_ref[:, 6:8] = jnp.zeros((_OUT_ROWS, 2), jnp.float32)

    fastf = kc_ref[1]

    @pl.when(fastf > 0.5)
    def _emit_fast():
        emit(_P1)

    @pl.when(fastf < 0.5)
    def _emit_slow():
        emit(_N)


def _nms(tabc, tabr, keyc, keyr):
    return pl.pallas_call(
        _nms_body,
        out_shape=jax.ShapeDtypeStruct((_OUT_ROWS, 8), jnp.float32),
        scratch_shapes=[
            pltpu.VMEM((_N, _N), jnp.bfloat16),
            pltpu.VMEM((_N, _N), jnp.bfloat16),
            pltpu.VMEM((8, _N), jnp.float32),
            pltpu.SMEM((2,), jnp.float32),
            pltpu.VMEM((_N, 8), jnp.float32),
            pltpu.VMEM((8, _N), jnp.float32),
            pltpu.VMEM((_N, _NC), jnp.bfloat16),
            pltpu.VMEM((1, _NC), jnp.float32),
        ],
    )(tabc, tabr, keyc, keyr)


def kernel(scores, bbox_deltas, im_info, anchors):
    A = anchors.shape[0]
    H, W = scores.shape[2], scores.shape[3]
    fg = scores[0, A:].reshape(A, H * W)
    dl = bbox_deltas[0].reshape(A, 4, H * W)

    x1, y1, x2, y2, scm, key, tsel = _transform(fg, dl, anchors, im_info)

    def flat(v):
        return v.T.reshape(-1)

    npad = _NALL - A * H * W
    k2 = jnp.concatenate([flat(key), jnp.full((npad,), -1, jnp.int32)])
    tab = jnp.stack([x1, y1, x2, y2, scm], axis=-1)
    tab = jnp.pad(tab, ((0, 0), (0, 0), (0, 11))).reshape(A * H * W, 16)
    tab = jnp.pad(tab, ((0, npad), (0, 0)))

    comp_tab, keys = _make_sc_topk()(k2, tsel, tab)
    tabc = comp_tab[:, :8]

    out = _nms(tabc, tabc.T, keys.reshape(_NCMP, 1), keys.reshape(1, _NCMP))
    return out[:POST_NMS_TOPN, :6]
